# Initial kernel scaffold; baseline (speedup 1.0000x reference)
import jax
import jax.numpy as jnp
from jax.experimental import pallas as pl


def kernel(node_features, edge_features, global_features, senders, receivers, edge_graph_ids, node_graph_ids, W_fe, b_fe, W_fs, b_fs, W_fr, b_fr, W_fu, b_fu, W_gn, b_gn, W_gin, b_gin, W_gout, b_gout, W_gu, b_gu, W_hn, b_hn, W_he, b_he, W_hu, b_hu):
    raise NotImplementedError("write your pallas kernel here")


# speedup vs baseline: 3.6616x; 3.6616x over previous
import functools

import jax
import jax.numpy as jnp
from jax import lax
from jax.experimental import pallas as pl
from jax.experimental.pallas import tpu as pltpu
from jax.experimental.pallas import tpu_sc as plsc

N = 10000
E = 320000
G = 8
D = 128
NC = 2
NS = 16
NW = NC * NS
CHUNK = 128
CHUNKS = E // CHUNK
CPW = -(-CHUNKS // NW)
RPT = 632
RPT_LAST = N - (NS - 1) * RPT
EBLK = 4000
NBLK = 2000
_P = lax.Precision.HIGHEST


def _dot(a, b):
    return jnp.dot(a, b, precision=_P, preferred_element_type=jnp.float32)


def _tables_body(nf_ref, gf_ref, wfs_ref, bfs_ref, wfr_ref, bfr_ref,
                 wfu_ref, bfu_ref, wgu_ref, bgu_ref,
                 fs_ref, fr_ref, fu_ref, gu_ref):
    nf = nf_ref[...]
    fs_ref[...] = _dot(nf, wfs_ref[...]) + bfs_ref[...]
    fr_ref[...] = _dot(nf, wfr_ref[...]) + bfr_ref[...]
    gf = gf_ref[...]
    fu_ref[...] = _dot(gf, wfu_ref[...]) + bfu_ref[...]
    gu_ref[...] = _dot(gf, wgu_ref[...]) + bgu_ref[...]


_tables_call = pl.pallas_call(
    _tables_body,
    out_shape=(
        jax.ShapeDtypeStruct((N, D), jnp.float32),
        jax.ShapeDtypeStruct((N, D), jnp.float32),
        jax.ShapeDtypeStruct((G, D), jnp.float32),
        jax.ShapeDtypeStruct((G, D), jnp.float32),
    ),
)


def _fe_body(ef_ref, gid_ref, fu_ref, wfe_ref, bfe_ref, out_ref):
    fe = _dot(ef_ref[...], wfe_ref[...]) + bfe_ref[...]
    gid = gid_ref[0, 0, :]
    onehot = (gid[:, None] == lax.broadcasted_iota(jnp.int32, (1, G), 1)
              ).astype(jnp.float32)
    out_ref[...] = fe + _dot(onehot, fu_ref[...])


_fe_call = pl.pallas_call(
    _fe_body,
    grid=(E // EBLK,),
    in_specs=[
        pl.BlockSpec((EBLK, 16), lambda i: (i, 0)),
        pl.BlockSpec((1, 1, EBLK), lambda i: (i, 0, 0)),
        pl.BlockSpec((G, D), lambda i: (0, 0)),
        pl.BlockSpec((16, D), lambda i: (0, 0)),
        pl.BlockSpec((1, D), lambda i: (0, 0)),
    ],
    out_specs=pl.BlockSpec((EBLK, D), lambda i: (i, 0)),
    out_shape=jax.ShapeDtypeStruct((E, D), jnp.float32),
)


_MESH = plsc.VectorSubcoreMesh(core_axis_name="c", subcore_axis_name="s",
                               num_cores=NC, num_subcores=NS)


def _acc_pieces(s, cb):
    start = s * RPT
    for j in range(4):
        cb(start + j * 128, 128)

    @pl.when(s < NS - 1)
    def _():
        cb(start + 512, RPT - 512)

    @pl.when(s == NS - 1)
    def _():
        cb(start + 512, RPT_LAST - 512)


@functools.partial(
    pl.kernel,
    out_type=(
        jax.ShapeDtypeStruct((E, D), jnp.float32),
        jax.ShapeDtypeStruct((NC * N, D), jnp.float32),
        jax.ShapeDtypeStruct((NC * NS * G, D), jnp.float32),
    ),
    mesh=_MESH,
    scratch_types=[
        pltpu.VMEM((CHUNK,), jnp.int32),
        pltpu.VMEM((CHUNK,), jnp.int32),
        pltpu.VMEM((CHUNK,), jnp.int32),
        pltpu.VMEM((CHUNK, D), jnp.float32),
        pltpu.VMEM((CHUNK, D), jnp.float32),
        pltpu.VMEM((CHUNK, D), jnp.float32),
        pltpu.VMEM_SHARED((N, D), jnp.float32),
        pltpu.VMEM_SHARED((NS * G, D), jnp.float32),
        pltpu.SemaphoreType.DMA,
        pltpu.SemaphoreType.DMA,
        pltpu.SemaphoreType.DMA,
    ],
)
def _sc_edge_pass(fe_hbm, fs_hbm, fr_hbm, snd_hbm, rcv_hbm, gid_hbm,
                  edges_hbm, aggin_hbm, pool_hbm,
                  sidx_v, ridx_v, pidx_v, fe_v, fs_v, fr_v,
                  acc_sh, pool_sh, sem1, sem2, sem3):
    c = lax.axis_index("c")
    s = lax.axis_index("s")
    wid = s * NC + c

    zv = jnp.zeros((16,), jnp.float32)

    def _zrow(r, carry):
        for j in range(D // 16):
            fe_v[r, pl.ds(j * 16, 16)] = zv
        return carry

    lax.fori_loop(0, CHUNK, _zrow, 0)
    _acc_pieces(s, lambda off, sz: pltpu.sync_copy(
        fe_v.at[pl.ds(0, sz)], acc_sh.at[pl.ds(off, sz)]))
    pltpu.sync_copy(fe_v.at[pl.ds(0, G)], pool_sh.at[pl.ds(s * G, G)])
    plsc.subcore_barrier()

    def _chunk(k, carry):
        cid = wid + k * NW

        @pl.when(cid < CHUNKS)
        def _():
            base = cid * CHUNK
            pltpu.sync_copy(snd_hbm.at[pl.ds(base, CHUNK)], sidx_v)
            pltpu.sync_copy(rcv_hbm.at[pl.ds(base, CHUNK)], ridx_v)
            pltpu.sync_copy(gid_hbm.at[pl.ds(base, CHUNK)], pidx_v)

            def _pfix(i, cy):
                sl = pl.ds(i * 16, 16)
                pidx_v[sl] = pidx_v[sl] + s * G
                return cy

            lax.fori_loop(0, CHUNK // 16, _pfix, 0)
            cp1 = pltpu.async_copy(fs_hbm.at[sidx_v], fs_v, sem1)
            cp2 = pltpu.async_copy(fr_hbm.at[ridx_v], fr_v, sem2)
            cp3 = pltpu.async_copy(fe_hbm.at[pl.ds(base, CHUNK)], fe_v, sem3)
            cp1.wait()
            cp2.wait()
            cp3.wait()

            def _crow(r, cy):
                for j in range(D // 16):
                    sl = pl.ds(j * 16, 16)
                    v = fe_v[r, sl] + fs_v[r, sl] + fr_v[r, sl]
                    fe_v[r, sl] = jnp.maximum(v, 0.0)
                return cy

            lax.fori_loop(0, CHUNK, _crow, 0)
            pltpu.sync_copy(fe_v, edges_hbm.at[pl.ds(base, CHUNK)])
            pltpu.sync_copy(fe_v, acc_sh.at[ridx_v], add=True)
            pltpu.sync_copy(fe_v, pool_sh.at[pidx_v], add=True)

        return carry

    lax.fori_loop(0, CPW, _chunk, 0)
    plsc.subcore_barrier()
    _acc_pieces(s, lambda off, sz: pltpu.sync_copy(
        acc_sh.at[pl.ds(off, sz)], aggin_hbm.at[pl.ds(c * N + off, sz)]))
    pltpu.sync_copy(pool_sh.at[pl.ds(s * G, G)],
                    pool_hbm.at[pl.ds((c * NS + s) * G, G)])


@functools.partial(
    pl.kernel,
    out_type=jax.ShapeDtypeStruct((NC * N, D), jnp.float32),
    mesh=_MESH,
    scratch_types=[
        pltpu.VMEM((CHUNK,), jnp.int32),
        pltpu.VMEM((CHUNK, D), jnp.float32),
        pltpu.VMEM_SHARED((N, D), jnp.float32),
        pltpu.SemaphoreType.DMA,
    ],
)
def _sc_aggout_pass(edges_hbm, snd_hbm, aggout_hbm,
                    sidx_v, ed_v, acc_sh, sem1):
    c = lax.axis_index("c")
    s = lax.axis_index("s")
    wid = s * NC + c

    zv = jnp.zeros((16,), jnp.float32)

    def _zrow(r, carry):
        for j in range(D // 16):
            ed_v[r, pl.ds(j * 16, 16)] = zv
        return carry

    lax.fori_loop(0, CHUNK, _zrow, 0)
    _acc_pieces(s, lambda off, sz: pltpu.sync_copy(
        ed_v.at[pl.ds(0, sz)], acc_sh.at[pl.ds(off, sz)]))
    plsc.subcore_barrier()

    def _chunk(k, carry):
        cid = wid + k * NW

        @pl.when(cid < CHUNKS)
        def _():
            base = cid * CHUNK
            pltpu.sync_copy(snd_hbm.at[pl.ds(base, CHUNK)], sidx_v)
            cp = pltpu.async_copy(edges_hbm.at[pl.ds(base, CHUNK)], ed_v, sem1)
            cp.wait()
            pltpu.sync_copy(ed_v, acc_sh.at[sidx_v], add=True)

        return carry

    lax.fori_loop(0, CPW, _chunk, 0)
    plsc.subcore_barrier()
    _acc_pieces(s, lambda off, sz: pltpu.sync_copy(
        acc_sh.at[pl.ds(off, sz)], aggout_hbm.at[pl.ds(c * N + off, sz)]))


def _node_body(nf_ref, ai0, ai1, ao0, ao1, gidn_ref, gu_ref, pool_ref,
               gf_ref, wgn, wgin, wgout, bn, whn, whe, whu, bh,
               nodes_ref, glob_ref, npool_ref):
    i = pl.program_id(0)
    agg_in = ai0[...] + ai1[...]
    agg_out = ao0[...] + ao1[...]
    x = (_dot(nf_ref[...], wgn[...]) + _dot(agg_in, wgin[...])
         + _dot(agg_out, wgout[...]) + bn[...])
    gid = gidn_ref[0, 0, :]
    onehot = (gid[:, None] == lax.broadcasted_iota(jnp.int32, (1, G), 1)
              ).astype(jnp.float32)
    x = x + _dot(onehot, gu_ref[...])
    nodes = jnp.maximum(x, 0.0)
    nodes_ref[...] = nodes
    onehot_t = (lax.broadcasted_iota(jnp.int32, (G, NBLK), 0) == gid[None, :]
                ).astype(jnp.float32)
    pp = _dot(onehot_t, nodes)

    @pl.when(i == 0)
    def _():
        npool_ref[...] = pp

    @pl.when(i > 0)
    def _():
        npool_ref[...] += pp

    @pl.when(i == pl.num_programs(0) - 1)
    def _():
        ep = pool_ref[pl.ds(0, G), :]
        for j in range(1, NC * NS):
            ep = ep + pool_ref[pl.ds(j * G, G), :]
        glob_ref[...] = (_dot(ep, whe[...]) + _dot(npool_ref[...], whn[...])
                         + _dot(gf_ref[...], whu[...]) + bh[...])


_node_call = pl.pallas_call(
    _node_body,
    grid=(N // NBLK,),
    in_specs=[
        pl.BlockSpec((NBLK, D), lambda i: (i, 0)),
        pl.BlockSpec((NBLK, D), lambda i: (i, 0)),
        pl.BlockSpec((NBLK, D), lambda i: (i + N // NBLK, 0)),
        pl.BlockSpec((NBLK, D), lambda i: (i, 0)),
        pl.BlockSpec((NBLK, D), lambda i: (i + N // NBLK, 0)),
        pl.BlockSpec((1, 1, NBLK), lambda i: (i, 0, 0)),
        pl.BlockSpec((G, D), lambda i: (0, 0)),
        pl.BlockSpec((NC * NS * G, D), lambda i: (0, 0)),
        pl.BlockSpec((G, 16), lambda i: (0, 0)),
        pl.BlockSpec((D, D), lambda i: (0, 0)),
        pl.BlockSpec((D, D), lambda i: (0, 0)),
        pl.BlockSpec((D, D), lambda i: (0, 0)),
        pl.BlockSpec((1, D), lambda i: (0, 0)),
        pl.BlockSpec((D, D), lambda i: (0, 0)),
        pl.BlockSpec((D, D), lambda i: (0, 0)),
        pl.BlockSpec((16, D), lambda i: (0, 0)),
        pl.BlockSpec((1, D), lambda i: (0, 0)),
    ],
    out_specs=(
        pl.BlockSpec((NBLK, D), lambda i: (i, 0)),
        pl.BlockSpec((G, D), lambda i: (0, 0)),
    ),
    out_shape=(
        jax.ShapeDtypeStruct((N, D), jnp.float32),
        jax.ShapeDtypeStruct((G, D), jnp.float32),
    ),
    scratch_shapes=[pltpu.VMEM((G, D), jnp.float32)],
)


def kernel(node_features, edge_features, global_features, senders, receivers,
           edge_graph_ids, node_graph_ids,
           W_fe, b_fe, W_fs, b_fs, W_fr, b_fr, W_fu, b_fu,
           W_gn, b_gn, W_gin, b_gin, W_gout, b_gout, W_gu, b_gu,
           W_hn, b_hn, W_he, b_he, W_hu, b_hu):
    fs_tab, fr_tab, fu_tab, gu_tab = _tables_call(
        node_features, global_features,
        W_fs, b_fs.reshape(1, D), W_fr, b_fr.reshape(1, D),
        W_fu, b_fu.reshape(1, D), W_gu, b_gu.reshape(1, D))
    fe_all = _fe_call(edge_features,
                      edge_graph_ids.reshape(E // EBLK, 1, EBLK),
                      fu_tab, W_fe, b_fe.reshape(1, D))
    edges, aggin_part, pool_part = _sc_edge_pass(
        fe_all, fs_tab, fr_tab, senders, receivers, edge_graph_ids)
    aggout_part = _sc_aggout_pass(edges, senders)
    bn_all = (b_gn + b_gin + b_gout).reshape(1, D)
    bh_all = (b_he + b_hn + b_hu).reshape(1, D)
    nodes, globals_out = _node_call(
        node_features, aggin_part, aggin_part, aggout_part, aggout_part,
        node_graph_ids.reshape(N // NBLK, 1, NBLK),
        gu_tab, pool_part, global_features,
        W_gn, W_gin, W_gout, bn_all, W_hn, W_he, W_hu, bh_all)
    return edges, nodes, globals_out


# speedup vs baseline: 4.9144x; 1.3421x over previous
import functools

import jax
import jax.numpy as jnp
from jax import lax
from jax.experimental import pallas as pl
from jax.experimental.pallas import tpu as pltpu
from jax.experimental.pallas import tpu_sc as plsc

N = 10000
E = 320000
G = 8
D = 128
NC = 2
NS = 16
NW = NC * NS
CA = 32
CHUNKS_A = E // CA
CPW_A = -(-CHUNKS_A // NW)
CB = 128
CHUNKS_B = E // CB
CPW_B = -(-CHUNKS_B // NW)
RPT = 632
RPT_LAST = N - (NS - 1) * RPT
EBLK = 4000
NBLK = 2000
_P = lax.Precision.HIGHEST


def _dot(a, b):
    return jnp.dot(a, b, precision=_P, preferred_element_type=jnp.float32)


def _tables_body(nf_ref, gf_ref, wfs_ref, bfs_ref, wfr_ref, bfr_ref,
                 wfu_ref, bfu_ref, wgu_ref, bgu_ref,
                 fs_ref, fr_ref, fu_ref, gu_ref):
    nf = nf_ref[...]
    fs_ref[...] = _dot(nf, wfs_ref[...]) + bfs_ref[...]
    fr_ref[...] = _dot(nf, wfr_ref[...]) + bfr_ref[...]
    gf = gf_ref[...]
    fu_ref[...] = _dot(gf, wfu_ref[...]) + bfu_ref[...]
    gu_ref[...] = _dot(gf, wgu_ref[...]) + bgu_ref[...]


_tables_call = pl.pallas_call(
    _tables_body,
    out_shape=(
        jax.ShapeDtypeStruct((N, D), jnp.float32),
        jax.ShapeDtypeStruct((N, D), jnp.float32),
        jax.ShapeDtypeStruct((G, D), jnp.float32),
        jax.ShapeDtypeStruct((G, D), jnp.float32),
    ),
)


def _fe_body(ef_ref, gid_ref, fu_ref, wfe_ref, bfe_ref, out_ref):
    fe = _dot(ef_ref[...], wfe_ref[...]) + bfe_ref[...]
    gid = gid_ref[0, 0, :]
    onehot = (gid[:, None] == lax.broadcasted_iota(jnp.int32, (1, G), 1)
              ).astype(jnp.float32)
    out_ref[...] = fe + _dot(onehot, fu_ref[...])


_fe_call = pl.pallas_call(
    _fe_body,
    grid=(E // EBLK,),
    in_specs=[
        pl.BlockSpec((EBLK, 16), lambda i: (i, 0)),
        pl.BlockSpec((1, 1, EBLK), lambda i: (i, 0, 0)),
        pl.BlockSpec((G, D), lambda i: (0, 0)),
        pl.BlockSpec((16, D), lambda i: (0, 0)),
        pl.BlockSpec((1, D), lambda i: (0, 0)),
    ],
    out_specs=pl.BlockSpec((EBLK, D), lambda i: (i, 0)),
    out_shape=jax.ShapeDtypeStruct((E, D), jnp.float32),
)


_MESH = plsc.VectorSubcoreMesh(core_axis_name="c", subcore_axis_name="s",
                               num_cores=NC, num_subcores=NS)


def _acc_pieces(s, cb):
    start = s * RPT
    for j in range(4):
        cb(start + j * 128, 128)

    @pl.when(s < NS - 1)
    def _():
        cb(start + 512, RPT - 512)

    @pl.when(s == NS - 1)
    def _():
        cb(start + 512, RPT_LAST - 512)


def _acc_pieces_small(s, cb):
    start = s * RPT
    for j in range(16):
        cb(start + j * 32, 32)

    @pl.when(s < NS - 1)
    def _():
        for j in range(3):
            cb(start + 512 + j * 32, 32)
        cb(start + 608, RPT - 608)

    @pl.when(s == NS - 1)
    def _():
        cb(start + 512, RPT_LAST - 512)


def _zero_buf(buf, rows):
    zv = jnp.zeros((16,), jnp.float32)

    def _zrow(r, carry):
        for j in range(D // 16):
            buf[r, pl.ds(j * 16, 16)] = zv
        return carry

    lax.fori_loop(0, rows, _zrow, 0)


@functools.partial(
    pl.kernel,
    out_type=(
        jax.ShapeDtypeStruct((E, D), jnp.float32),
        jax.ShapeDtypeStruct((NC * N, D), jnp.float32),
    ),
    mesh=_MESH,
    scratch_types=(
        [pltpu.VMEM((CA,), jnp.int32)] * 8
        + [pltpu.VMEM((CA, D), jnp.float32)] * 8
        + [pltpu.VMEM_SHARED((N, D), jnp.float32)]
        + [pltpu.SemaphoreType.DMA] * 12
    ),
)
def _sc_edge_pass(fe_hbm, fs_hbm, fr_hbm, snd_hbm, rcv_hbm,
                  edges_hbm, aggin_hbm,
                  si0, ri0, si1, ri1, si2, ri2, si3, ri3,
                  fs0, fr0, fe0, ou0, fs1, fr1, fe1, ou1,
                  acc_sh,
                  smi0, smi1, smi2, smi3, smg0, smg1, smf0, smf1,
                  sms0, sms1, sma0, sma1):
    c = lax.axis_index("c")
    s = lax.axis_index("s")
    wid = s * NC + c
    isets = [(si0, ri0, smi0), (si1, ri1, smi1),
             (si2, ri2, smi2), (si3, ri3, smi3)]
    dsets = [(fs0, fr0, fe0, ou0, smg0, smf0, sms0, sma0),
             (fs1, fr1, fe1, ou1, smg1, smf1, sms1, sma1)]

    _zero_buf(fe0, CA)
    _acc_pieces_small(s, lambda off, sz: pltpu.sync_copy(
        fe0.at[pl.ds(0, sz)], acc_sh.at[pl.ds(off, sz)]))
    plsc.subcore_barrier()

    def cid_of(j):
        return wid + j * NW

    def issue_idx(j, iset):
        sidx, ridx, smi = iset

        @pl.when(cid_of(j) < CHUNKS_A)
        def _():
            base = cid_of(j) * CA
            pltpu.async_copy(snd_hbm.at[pl.ds(base, CA)], sidx, smi)
            pltpu.async_copy(rcv_hbm.at[pl.ds(base, CA)], ridx, smi)

    def prefetch_data(j, iset, dset):
        sidx, ridx, smi = iset
        fs_b, fr_b, fe_b, _, smg, smf, _, _ = dset

        @pl.when(cid_of(j) < CHUNKS_A)
        def _():
            for _ in range(2):
                pltpu.make_async_copy(
                    snd_hbm.at[pl.ds(0, CA)], sidx, smi).wait()
            pltpu.async_copy(fs_hbm.at[sidx], fs_b, smg)
            pltpu.async_copy(fr_hbm.at[ridx], fr_b, smg)
            pltpu.async_copy(fe_hbm.at[pl.ds(cid_of(j) * CA, CA)],
                             fe_b, smf)

    def step(k, icur, invt, inxt, dcur, dnxt):
        sidx, ridx, _ = icur
        fs_b, fr_b, fe_b, out_b, smg, smf, sms, sma = dcur
        valid_k = cid_of(k) < CHUNKS_A

        prefetch_data(k + 1, inxt, dnxt)
        issue_idx(k + 2, invt)

        @pl.when(valid_k)
        def _():
            pltpu.make_async_copy(fs_hbm.at[sidx], fs_b, smg).wait()
            pltpu.make_async_copy(fr_hbm.at[ridx], fr_b, smg).wait()
            pltpu.make_async_copy(
                fe_hbm.at[pl.ds(0, CA)], fe_b, smf).wait()

            def _crow(r, cy):
                for rr in range(2):
                    for j in range(D // 16):
                        sl = pl.ds(j * 16, 16)
                        v = (fe_b[2 * r + rr, sl] + fs_b[2 * r + rr, sl]
                             + fr_b[2 * r + rr, sl])
                        out_b[2 * r + rr, sl] = jnp.maximum(v, 0.0)
                return cy

            lax.fori_loop(0, CA // 2, _crow, 0)

        _, _, _, outn, _, _, smsn, sman = dnxt
        _, rin, _ = inxt

        @pl.when((k >= 1) & (cid_of(k - 1) < CHUNKS_A))
        def _():
            pltpu.make_async_copy(
                outn, edges_hbm.at[pl.ds(0, CA)], smsn).wait()
            pltpu.make_async_copy(outn, acc_sh.at[rin], sman).wait()

        @pl.when(valid_k)
        def _():
            base = cid_of(k) * CA
            pltpu.async_copy(out_b, edges_hbm.at[pl.ds(base, CA)], sms)
            pltpu.async_copy(out_b, acc_sh.at[ridx], sma, add=True)

    issue_idx(0, isets[0])
    issue_idx(1, isets[1])
    prefetch_data(0, isets[0], dsets[0])

    def _quad(t, carry):
        k = 4 * t
        step(k, isets[0], isets[2], isets[1], dsets[0], dsets[1])
        step(k + 1, isets[1], isets[3], isets[2], dsets[1], dsets[0])
        step(k + 2, isets[2], isets[0], isets[3], dsets[0], dsets[1])
        step(k + 3, isets[3], isets[1], isets[0], dsets[1], dsets[0])
        return carry

    lax.fori_loop(0, (CPW_A + 4) // 4, _quad, 0)
    kl = (CPW_A + 4) // 4 * 4 - 1

    def _drain_tail(j, dset, iset):
        _, _, _, outt, _, _, smst, smat = dset
        _, rit, _ = iset

        @pl.when(cid_of(j) < CHUNKS_A)
        def _():
            pltpu.make_async_copy(
                outt, edges_hbm.at[pl.ds(0, CA)], smst).wait()
            pltpu.make_async_copy(outt, acc_sh.at[rit], smat).wait()

    _drain_tail(kl, dsets[kl % 2], isets[kl % 4])
    plsc.subcore_barrier()
    _acc_pieces(s, lambda off, sz: pltpu.sync_copy(
        acc_sh.at[pl.ds(off, sz)], aggin_hbm.at[pl.ds(c * N + off, sz)]))


@functools.partial(
    pl.kernel,
    out_type=(
        jax.ShapeDtypeStruct((NC * N, D), jnp.float32),
        jax.ShapeDtypeStruct((NC * NS * G, D), jnp.float32),
    ),
    mesh=_MESH,
    scratch_types=(
        [pltpu.VMEM((CB,), jnp.int32)] * 6
        + [pltpu.VMEM((CB, D), jnp.float32)] * 3
        + [pltpu.VMEM_SHARED((N, D), jnp.float32),
           pltpu.VMEM_SHARED((NS * G, D), jnp.float32)]
        + [pltpu.SemaphoreType.DMA] * 8
    ),
)
def _sc_aggout_pass(edges_hbm, snd_hbm, gid_hbm, aggout_hbm, pool_hbm,
                    si0, pi0, si1, pi1, si2, pi2,
                    ed0, ed1, ed2, acc_sh, pool_sh,
                    smi0, smi1, smi2, smf0, smf1, smf2, sms0, sms1):
    c = lax.axis_index("c")
    s = lax.axis_index("s")
    wid = s * NC + c
    isets = [(si0, pi0, smi0), (si1, pi1, smi1), (si2, pi2, smi2)]
    dsets = [(ed0, smf0), (ed1, smf1), (ed2, smf2)]

    _zero_buf(ed0, CB)
    _acc_pieces(s, lambda off, sz: pltpu.sync_copy(
        ed0.at[pl.ds(0, sz)], acc_sh.at[pl.ds(off, sz)]))
    pltpu.sync_copy(ed0.at[pl.ds(0, G)], pool_sh.at[pl.ds(s * G, G)])
    plsc.subcore_barrier()

    def cid_of(j):
        return wid + j * NW

    def issue_idx(j, iset):
        sidx, pidx, smi = iset

        @pl.when(cid_of(j) < CHUNKS_B)
        def _():
            base = cid_of(j) * CB
            pltpu.async_copy(snd_hbm.at[pl.ds(base, CB)], sidx, smi)
            pltpu.async_copy(gid_hbm.at[pl.ds(base, CB)], pidx, smi)

    def prefetch_data(j, dset):
        ed_b, smf = dset

        @pl.when(cid_of(j) < CHUNKS_B)
        def _():
            pltpu.async_copy(
                edges_hbm.at[pl.ds(cid_of(j) * CB, CB)], ed_b, smf)

    def drain_adds(pred, sms):
        @pl.when(pred)
        def _():
            pltpu.make_async_copy(ed0, acc_sh.at[si0], sms).wait()
            pltpu.make_async_copy(ed0, pool_sh.at[pi0], sms).wait()

    def step(k, icur, invt, dcur, dnxt, sms, smsn):
        sidx, pidx, smi = icur
        ed_b, smf = dcur

        prefetch_data(k + 1, dnxt)

        @pl.when(cid_of(k) < CHUNKS_B)
        def _():
            pltpu.make_async_copy(
                edges_hbm.at[pl.ds(0, CB)], ed_b, smf).wait()
            for _ in range(2):
                pltpu.make_async_copy(
                    snd_hbm.at[pl.ds(0, CB)], sidx, smi).wait()
            for i in range(CB // 16):
                sl = pl.ds(i * 16, 16)
                pidx[sl] = pidx[sl] + s * G
            pltpu.async_copy(ed_b, acc_sh.at[sidx], sms, add=True)
            pltpu.async_copy(ed_b, pool_sh.at[pidx], sms, add=True)

        drain_adds((k >= 1) & (cid_of(k - 1) < CHUNKS_B), smsn)
        issue_idx(k + 2, invt)

    issue_idx(0, isets[0])
    issue_idx(1, isets[1])
    prefetch_data(0, dsets[0])

    smss = [sms0, sms1]

    def _hex(t, carry):
        k = 6 * t
        for u in range(6):
            step(k + u, isets[u % 3], isets[(u + 2) % 3],
                 dsets[u % 3], dsets[(u + 1) % 3],
                 smss[u % 2], smss[(u + 1) % 2])
        return carry

    lax.fori_loop(0, (CPW_B + 6) // 6, _hex, 0)
    plsc.subcore_barrier()
    _acc_pieces(s, lambda off, sz: pltpu.sync_copy(
        acc_sh.at[pl.ds(off, sz)], aggout_hbm.at[pl.ds(c * N + off, sz)]))
    pltpu.sync_copy(pool_sh.at[pl.ds(s * G, G)],
                    pool_hbm.at[pl.ds((c * NS + s) * G, G)])


def _node_body(nf_ref, ai0, ai1, ao0, ao1, gidn_ref, gu_ref, pool_ref,
               gf_ref, wgn, wgin, wgout, bn, whn, whe, whu, bh,
               nodes_ref, glob_ref, npool_ref):
    i = pl.program_id(0)
    agg_in = ai0[...] + ai1[...]
    agg_out = ao0[...] + ao1[...]
    x = (_dot(nf_ref[...], wgn[...]) + _dot(agg_in, wgin[...])
         + _dot(agg_out, wgout[...]) + bn[...])
    gid = gidn_ref[0, 0, :]
    onehot = (gid[:, None] == lax.broadcasted_iota(jnp.int32, (1, G), 1)
              ).astype(jnp.float32)
    x = x + _dot(onehot, gu_ref[...])
    nodes = jnp.maximum(x, 0.0)
    nodes_ref[...] = nodes
    onehot_t = (lax.broadcasted_iota(jnp.int32, (G, NBLK), 0) == gid[None, :]
                ).astype(jnp.float32)
    pp = _dot(onehot_t, nodes)

    @pl.when(i == 0)
    def _():
        npool_ref[...] = pp

    @pl.when(i > 0)
    def _():
        npool_ref[...] += pp

    @pl.when(i == pl.num_programs(0) - 1)
    def _():
        ep = pool_ref[pl.ds(0, G), :]
        for j in range(1, NC * NS):
            ep = ep + pool_ref[pl.ds(j * G, G), :]
        glob_ref[...] = (_dot(ep, whe[...]) + _dot(npool_ref[...], whn[...])
                         + _dot(gf_ref[...], whu[...]) + bh[...])


_node_call = pl.pallas_call(
    _node_body,
    grid=(N // NBLK,),
    in_specs=[
        pl.BlockSpec((NBLK, D), lambda i: (i, 0)),
        pl.BlockSpec((NBLK, D), lambda i: (i, 0)),
        pl.BlockSpec((NBLK, D), lambda i: (i + N // NBLK, 0)),
        pl.BlockSpec((NBLK, D), lambda i: (i, 0)),
        pl.BlockSpec((NBLK, D), lambda i: (i + N // NBLK, 0)),
        pl.BlockSpec((1, 1, NBLK), lambda i: (i, 0, 0)),
        pl.BlockSpec((G, D), lambda i: (0, 0)),
        pl.BlockSpec((NC * NS * G, D), lambda i: (0, 0)),
        pl.BlockSpec((G, 16), lambda i: (0, 0)),
        pl.BlockSpec((D, D), lambda i: (0, 0)),
        pl.BlockSpec((D, D), lambda i: (0, 0)),
        pl.BlockSpec((D, D), lambda i: (0, 0)),
        pl.BlockSpec((1, D), lambda i: (0, 0)),
        pl.BlockSpec((D, D), lambda i: (0, 0)),
        pl.BlockSpec((D, D), lambda i: (0, 0)),
        pl.BlockSpec((16, D), lambda i: (0, 0)),
        pl.BlockSpec((1, D), lambda i: (0, 0)),
    ],
    out_specs=(
        pl.BlockSpec((NBLK, D), lambda i: (i, 0)),
        pl.BlockSpec((G, D), lambda i: (0, 0)),
    ),
    out_shape=(
        jax.ShapeDtypeStruct((N, D), jnp.float32),
        jax.ShapeDtypeStruct((G, D), jnp.float32),
    ),
    scratch_shapes=[pltpu.VMEM((G, D), jnp.float32)],
)


def kernel(node_features, edge_features, global_features, senders, receivers,
           edge_graph_ids, node_graph_ids,
           W_fe, b_fe, W_fs, b_fs, W_fr, b_fr, W_fu, b_fu,
           W_gn, b_gn, W_gin, b_gin, W_gout, b_gout, W_gu, b_gu,
           W_hn, b_hn, W_he, b_he, W_hu, b_hu):
    fs_tab, fr_tab, fu_tab, gu_tab = _tables_call(
        node_features, global_features,
        W_fs, b_fs.reshape(1, D), W_fr, b_fr.reshape(1, D),
        W_fu, b_fu.reshape(1, D), W_gu, b_gu.reshape(1, D))
    fe_all = _fe_call(edge_features,
                      edge_graph_ids.reshape(E // EBLK, 1, EBLK),
                      fu_tab, W_fe, b_fe.reshape(1, D))
    edges, aggin_part = _sc_edge_pass(
        fe_all, fs_tab, fr_tab, senders, receivers)
    aggout_part, pool_part = _sc_aggout_pass(edges, senders, edge_graph_ids)
    bn_all = (b_gn + b_gin + b_gout).reshape(1, D)
    bh_all = (b_he + b_hn + b_hu).reshape(1, D)
    nodes, globals_out = _node_call(
        node_features, aggin_part, aggin_part, aggout_part, aggout_part,
        node_graph_ids.reshape(N // NBLK, 1, NBLK),
        gu_tab, pool_part, global_features,
        W_gn, W_gin, W_gout, bn_all, W_hn, W_he, W_hu, bh_all)
    return edges, nodes, globals_out


# speedup vs baseline: 6.1259x; 1.2465x over previous
import functools

import jax
import jax.numpy as jnp
from jax import lax
from jax.experimental import pallas as pl
from jax.experimental.pallas import tpu as pltpu
from jax.experimental.pallas import tpu_sc as plsc

N = 10000
E = 320000
G = 8
D = 128
NC = 2
NS = 16
NW = NC * NS
CA = 32
CHUNKS_A = E // CA
CPW_A = -(-CHUNKS_A // NW)
CB = 128
CHUNKS_B = E // CB
CPW_B = -(-CHUNKS_B // NW)
RPT = 632
RPT_LAST = N - (NS - 1) * RPT
EBLK = 4000
NBLK = 2000
def _dot(a, b):
    return jnp.dot(a, b, preferred_element_type=jnp.float32)


def _tables_body(nf_ref, gf_ref, wfs_ref, bfs_ref, wfr_ref, bfr_ref,
                 wfu_ref, bfu_ref, wgu_ref, bgu_ref,
                 fs_ref, fr_ref, fu_ref, gu_ref):
    nf = nf_ref[...]
    fs_ref[...] = _dot(nf, wfs_ref[...]) + bfs_ref[...]
    fr_ref[...] = _dot(nf, wfr_ref[...]) + bfr_ref[...]
    gf = gf_ref[...]
    fu_ref[...] = _dot(gf, wfu_ref[...]) + bfu_ref[...]
    gu_ref[...] = _dot(gf, wgu_ref[...]) + bgu_ref[...]


_tables_call = pl.pallas_call(
    _tables_body,
    out_shape=(
        jax.ShapeDtypeStruct((N, D), jnp.float32),
        jax.ShapeDtypeStruct((N, D), jnp.float32),
        jax.ShapeDtypeStruct((G, D), jnp.float32),
        jax.ShapeDtypeStruct((G, D), jnp.float32),
    ),
)


def _fe_body(ef_ref, gid_ref, fu_ref, wfe_ref, bfe_ref, out_ref):
    fe = _dot(ef_ref[...], wfe_ref[...]) + bfe_ref[...]
    gid = gid_ref[0, 0, :]
    onehot = (gid[:, None] == lax.broadcasted_iota(jnp.int32, (1, G), 1)
              ).astype(jnp.float32)
    out_ref[...] = fe + _dot(onehot, fu_ref[...])


_fe_call = pl.pallas_call(
    _fe_body,
    grid=(E // EBLK,),
    in_specs=[
        pl.BlockSpec((EBLK, 16), lambda i: (i, 0)),
        pl.BlockSpec((1, 1, EBLK), lambda i: (i, 0, 0)),
        pl.BlockSpec((G, D), lambda i: (0, 0)),
        pl.BlockSpec((16, D), lambda i: (0, 0)),
        pl.BlockSpec((1, D), lambda i: (0, 0)),
    ],
    out_specs=pl.BlockSpec((EBLK, D), lambda i: (i, 0)),
    out_shape=jax.ShapeDtypeStruct((E, D), jnp.float32),
)


_MESH = plsc.VectorSubcoreMesh(core_axis_name="c", subcore_axis_name="s",
                               num_cores=NC, num_subcores=NS)


def _acc_pieces(s, cb):
    start = s * RPT
    for j in range(4):
        cb(start + j * 128, 128)

    @pl.when(s < NS - 1)
    def _():
        cb(start + 512, RPT - 512)

    @pl.when(s == NS - 1)
    def _():
        cb(start + 512, RPT_LAST - 512)


def _acc_pieces_small(s, cb):
    start = s * RPT
    for j in range(16):
        cb(start + j * 32, 32)

    @pl.when(s < NS - 1)
    def _():
        for j in range(3):
            cb(start + 512 + j * 32, 32)
        cb(start + 608, RPT - 608)

    @pl.when(s == NS - 1)
    def _():
        cb(start + 512, RPT_LAST - 512)


def _zero_buf(buf, rows):
    zv = jnp.zeros((16,), jnp.float32)

    def _zrow(r, carry):
        for j in range(D // 16):
            buf[r, pl.ds(j * 16, 16)] = zv
        return carry

    lax.fori_loop(0, rows, _zrow, 0)


@functools.partial(
    pl.kernel,
    out_type=(
        jax.ShapeDtypeStruct((E, D), jnp.float32),
        jax.ShapeDtypeStruct((NC * N, D), jnp.float32),
    ),
    mesh=_MESH,
    scratch_types=(
        [pltpu.VMEM((CA,), jnp.int32)] * 8
        + [pltpu.VMEM((CA, D), jnp.float32)] * 8
        + [pltpu.VMEM_SHARED((N, D), jnp.float32)]
        + [pltpu.SemaphoreType.DMA] * 12
    ),
)
def _sc_edge_pass(fe_hbm, fs_hbm, fr_hbm, snd_hbm, rcv_hbm,
                  edges_hbm, aggin_hbm,
                  si0, ri0, si1, ri1, si2, ri2, si3, ri3,
                  fs0, fr0, fe0, ou0, fs1, fr1, fe1, ou1,
                  acc_sh,
                  smi0, smi1, smi2, smi3, smg0, smg1, smf0, smf1,
                  sms0, sms1, sma0, sma1):
    c = lax.axis_index("c")
    s = lax.axis_index("s")
    wid = s * NC + c
    isets = [(si0, ri0, smi0), (si1, ri1, smi1),
             (si2, ri2, smi2), (si3, ri3, smi3)]
    dsets = [(fs0, fr0, fe0, ou0, smg0, smf0, sms0, sma0),
             (fs1, fr1, fe1, ou1, smg1, smf1, sms1, sma1)]

    _zero_buf(fe0, CA)
    _acc_pieces_small(s, lambda off, sz: pltpu.sync_copy(
        fe0.at[pl.ds(0, sz)], acc_sh.at[pl.ds(off, sz)]))
    plsc.subcore_barrier()

    def cid_of(j):
        return wid + j * NW

    def issue_idx(j, iset):
        sidx, ridx, smi = iset

        @pl.when(cid_of(j) < CHUNKS_A)
        def _():
            base = cid_of(j) * CA
            pltpu.async_copy(snd_hbm.at[pl.ds(base, CA)], sidx, smi)
            pltpu.async_copy(rcv_hbm.at[pl.ds(base, CA)], ridx, smi)

    def prefetch_data(j, iset, dset):
        sidx, ridx, smi = iset
        fs_b, fr_b, fe_b, _, smg, smf, _, _ = dset

        @pl.when(cid_of(j) < CHUNKS_A)
        def _():
            for _ in range(2):
                pltpu.make_async_copy(
                    snd_hbm.at[pl.ds(0, CA)], sidx, smi).wait()
            pltpu.async_copy(fs_hbm.at[sidx], fs_b, smg)
            pltpu.async_copy(fr_hbm.at[ridx], fr_b, smg)
            pltpu.async_copy(fe_hbm.at[pl.ds(cid_of(j) * CA, CA)],
                             fe_b, smf)

    def step(k, icur, invt, inxt, dcur, dnxt):
        sidx, ridx, _ = icur
        fs_b, fr_b, fe_b, out_b, smg, smf, sms, sma = dcur
        valid_k = cid_of(k) < CHUNKS_A

        prefetch_data(k + 1, inxt, dnxt)
        issue_idx(k + 2, invt)

        @pl.when(valid_k)
        def _():
            pltpu.make_async_copy(fs_hbm.at[sidx], fs_b, smg).wait()
            pltpu.make_async_copy(fr_hbm.at[ridx], fr_b, smg).wait()
            pltpu.make_async_copy(
                fe_hbm.at[pl.ds(0, CA)], fe_b, smf).wait()

            def _crow(r, cy):
                for rr in range(2):
                    for j in range(D // 16):
                        sl = pl.ds(j * 16, 16)
                        v = (fe_b[2 * r + rr, sl] + fs_b[2 * r + rr, sl]
                             + fr_b[2 * r + rr, sl])
                        out_b[2 * r + rr, sl] = jnp.maximum(v, 0.0)
                return cy

            lax.fori_loop(0, CA // 2, _crow, 0)

        _, _, _, outn, _, _, smsn, sman = dnxt
        _, rin, _ = inxt

        @pl.when((k >= 1) & (cid_of(k - 1) < CHUNKS_A))
        def _():
            pltpu.make_async_copy(
                outn, edges_hbm.at[pl.ds(0, CA)], smsn).wait()
            pltpu.make_async_copy(outn, acc_sh.at[rin], sman).wait()

        @pl.when(valid_k)
        def _():
            base = cid_of(k) * CA
            pltpu.async_copy(out_b, edges_hbm.at[pl.ds(base, CA)], sms)
            pltpu.async_copy(out_b, acc_sh.at[ridx], sma, add=True)

    issue_idx(0, isets[0])
    issue_idx(1, isets[1])
    prefetch_data(0, isets[0], dsets[0])

    def _quad(t, carry):
        k = 4 * t
        step(k, isets[0], isets[2], isets[1], dsets[0], dsets[1])
        step(k + 1, isets[1], isets[3], isets[2], dsets[1], dsets[0])
        step(k + 2, isets[2], isets[0], isets[3], dsets[0], dsets[1])
        step(k + 3, isets[3], isets[1], isets[0], dsets[1], dsets[0])
        return carry

    lax.fori_loop(0, (CPW_A + 4) // 4, _quad, 0)
    kl = (CPW_A + 4) // 4 * 4 - 1

    def _drain_tail(j, dset, iset):
        _, _, _, outt, _, _, smst, smat = dset
        _, rit, _ = iset

        @pl.when(cid_of(j) < CHUNKS_A)
        def _():
            pltpu.make_async_copy(
                outt, edges_hbm.at[pl.ds(0, CA)], smst).wait()
            pltpu.make_async_copy(outt, acc_sh.at[rit], smat).wait()

    _drain_tail(kl, dsets[kl % 2], isets[kl % 4])
    plsc.subcore_barrier()
    _acc_pieces(s, lambda off, sz: pltpu.sync_copy(
        acc_sh.at[pl.ds(off, sz)], aggin_hbm.at[pl.ds(c * N + off, sz)]))


@functools.partial(
    pl.kernel,
    out_type=(
        jax.ShapeDtypeStruct((NC * N, D), jnp.float32),
        jax.ShapeDtypeStruct((NC * NS * G, D), jnp.float32),
    ),
    mesh=_MESH,
    scratch_types=(
        [pltpu.VMEM((CB,), jnp.int32)] * 6
        + [pltpu.VMEM((CB, D), jnp.float32)] * 3
        + [pltpu.VMEM_SHARED((N, D), jnp.float32),
           pltpu.VMEM_SHARED((NS * G, D), jnp.float32)]
        + [pltpu.SemaphoreType.DMA] * 8
    ),
)
def _sc_aggout_pass(edges_hbm, snd_hbm, gid_hbm, aggout_hbm, pool_hbm,
                    si0, pi0, si1, pi1, si2, pi2,
                    ed0, ed1, ed2, acc_sh, pool_sh,
                    smi0, smi1, smi2, smf0, smf1, smf2, sms0, sms1):
    c = lax.axis_index("c")
    s = lax.axis_index("s")
    wid = s * NC + c
    isets = [(si0, pi0, smi0), (si1, pi1, smi1), (si2, pi2, smi2)]
    dsets = [(ed0, smf0), (ed1, smf1), (ed2, smf2)]

    _zero_buf(ed0, CB)
    _acc_pieces(s, lambda off, sz: pltpu.sync_copy(
        ed0.at[pl.ds(0, sz)], acc_sh.at[pl.ds(off, sz)]))
    pltpu.sync_copy(ed0.at[pl.ds(0, G)], pool_sh.at[pl.ds(s * G, G)])
    plsc.subcore_barrier()

    def cid_of(j):
        return wid + j * NW

    def issue_idx(j, iset):
        sidx, pidx, smi = iset

        @pl.when(cid_of(j) < CHUNKS_B)
        def _():
            base = cid_of(j) * CB
            pltpu.async_copy(snd_hbm.at[pl.ds(base, CB)], sidx, smi)
            pltpu.async_copy(gid_hbm.at[pl.ds(base, CB)], pidx, smi)

    def prefetch_data(j, dset):
        ed_b, smf = dset

        @pl.when(cid_of(j) < CHUNKS_B)
        def _():
            pltpu.async_copy(
                edges_hbm.at[pl.ds(cid_of(j) * CB, CB)], ed_b, smf)

    def drain_adds(pred, sms):
        @pl.when(pred)
        def _():
            pltpu.make_async_copy(ed0, acc_sh.at[si0], sms).wait()
            pltpu.make_async_copy(ed0, pool_sh.at[pi0], sms).wait()

    def step(k, icur, invt, dcur, dnxt, sms, smsn):
        sidx, pidx, smi = icur
        ed_b, smf = dcur

        prefetch_data(k + 1, dnxt)

        @pl.when(cid_of(k) < CHUNKS_B)
        def _():
            pltpu.make_async_copy(
                edges_hbm.at[pl.ds(0, CB)], ed_b, smf).wait()
            for _ in range(2):
                pltpu.make_async_copy(
                    snd_hbm.at[pl.ds(0, CB)], sidx, smi).wait()
            for i in range(CB // 16):
                sl = pl.ds(i * 16, 16)
                pidx[sl] = pidx[sl] + s * G
            pltpu.async_copy(ed_b, acc_sh.at[sidx], sms, add=True)
            pltpu.async_copy(ed_b, pool_sh.at[pidx], sms, add=True)

        drain_adds((k >= 1) & (cid_of(k - 1) < CHUNKS_B), smsn)
        issue_idx(k + 2, invt)

    issue_idx(0, isets[0])
    issue_idx(1, isets[1])
    prefetch_data(0, dsets[0])

    smss = [sms0, sms1]

    def _hex(t, carry):
        k = 6 * t
        for u in range(6):
            step(k + u, isets[u % 3], isets[(u + 2) % 3],
                 dsets[u % 3], dsets[(u + 1) % 3],
                 smss[u % 2], smss[(u + 1) % 2])
        return carry

    lax.fori_loop(0, (CPW_B + 6) // 6, _hex, 0)
    plsc.subcore_barrier()
    _acc_pieces(s, lambda off, sz: pltpu.sync_copy(
        acc_sh.at[pl.ds(off, sz)], aggout_hbm.at[pl.ds(c * N + off, sz)]))
    pltpu.sync_copy(pool_sh.at[pl.ds(s * G, G)],
                    pool_hbm.at[pl.ds((c * NS + s) * G, G)])


def _node_body(nf_ref, ai0, ai1, ao0, ao1, gidn_ref, gu_ref, pool_ref,
               gf_ref, wgn, wgin, wgout, bn, whn, whe, whu, bh,
               nodes_ref, glob_ref, npool_ref):
    i = pl.program_id(0)
    agg_in = ai0[...] + ai1[...]
    agg_out = ao0[...] + ao1[...]
    x = (_dot(nf_ref[...], wgn[...]) + _dot(agg_in, wgin[...])
         + _dot(agg_out, wgout[...]) + bn[...])
    gid = gidn_ref[0, 0, :]
    onehot = (gid[:, None] == lax.broadcasted_iota(jnp.int32, (1, G), 1)
              ).astype(jnp.float32)
    x = x + _dot(onehot, gu_ref[...])
    nodes = jnp.maximum(x, 0.0)
    nodes_ref[...] = nodes
    onehot_t = (lax.broadcasted_iota(jnp.int32, (G, NBLK), 0) == gid[None, :]
                ).astype(jnp.float32)
    pp = _dot(onehot_t, nodes)

    @pl.when(i == 0)
    def _():
        npool_ref[...] = pp

    @pl.when(i > 0)
    def _():
        npool_ref[...] += pp

    @pl.when(i == pl.num_programs(0) - 1)
    def _():
        ep = pool_ref[pl.ds(0, G), :]
        for j in range(1, NC * NS):
            ep = ep + pool_ref[pl.ds(j * G, G), :]
        glob_ref[...] = (_dot(ep, whe[...]) + _dot(npool_ref[...], whn[...])
                         + _dot(gf_ref[...], whu[...]) + bh[...])


_node_call = pl.pallas_call(
    _node_body,
    grid=(N // NBLK,),
    in_specs=[
        pl.BlockSpec((NBLK, D), lambda i: (i, 0)),
        pl.BlockSpec((NBLK, D), lambda i: (i, 0)),
        pl.BlockSpec((NBLK, D), lambda i: (i + N // NBLK, 0)),
        pl.BlockSpec((NBLK, D), lambda i: (i, 0)),
        pl.BlockSpec((NBLK, D), lambda i: (i + N // NBLK, 0)),
        pl.BlockSpec((1, 1, NBLK), lambda i: (i, 0, 0)),
        pl.BlockSpec((G, D), lambda i: (0, 0)),
        pl.BlockSpec((NC * NS * G, D), lambda i: (0, 0)),
        pl.BlockSpec((G, 16), lambda i: (0, 0)),
        pl.BlockSpec((D, D), lambda i: (0, 0)),
        pl.BlockSpec((D, D), lambda i: (0, 0)),
        pl.BlockSpec((D, D), lambda i: (0, 0)),
        pl.BlockSpec((1, D), lambda i: (0, 0)),
        pl.BlockSpec((D, D), lambda i: (0, 0)),
        pl.BlockSpec((D, D), lambda i: (0, 0)),
        pl.BlockSpec((16, D), lambda i: (0, 0)),
        pl.BlockSpec((1, D), lambda i: (0, 0)),
    ],
    out_specs=(
        pl.BlockSpec((NBLK, D), lambda i: (i, 0)),
        pl.BlockSpec((G, D), lambda i: (0, 0)),
    ),
    out_shape=(
        jax.ShapeDtypeStruct((N, D), jnp.float32),
        jax.ShapeDtypeStruct((G, D), jnp.float32),
    ),
    scratch_shapes=[pltpu.VMEM((G, D), jnp.float32)],
)


def kernel(node_features, edge_features, global_features, senders, receivers,
           edge_graph_ids, node_graph_ids,
           W_fe, b_fe, W_fs, b_fs, W_fr, b_fr, W_fu, b_fu,
           W_gn, b_gn, W_gin, b_gin, W_gout, b_gout, W_gu, b_gu,
           W_hn, b_hn, W_he, b_he, W_hu, b_hu):
    fs_tab, fr_tab, fu_tab, gu_tab = _tables_call(
        node_features, global_features,
        W_fs, b_fs.reshape(1, D), W_fr, b_fr.reshape(1, D),
        W_fu, b_fu.reshape(1, D), W_gu, b_gu.reshape(1, D))
    fe_all = _fe_call(edge_features,
                      edge_graph_ids.reshape(E // EBLK, 1, EBLK),
                      fu_tab, W_fe, b_fe.reshape(1, D))
    edges, aggin_part = _sc_edge_pass(
        fe_all, fs_tab, fr_tab, senders, receivers)
    aggout_part, pool_part = _sc_aggout_pass(edges, senders, edge_graph_ids)
    bn_all = (b_gn + b_gin + b_gout).reshape(1, D)
    bh_all = (b_he + b_hn + b_hu).reshape(1, D)
    nodes, globals_out = _node_call(
        node_features, aggin_part, aggin_part, aggout_part, aggout_part,
        node_graph_ids.reshape(N // NBLK, 1, NBLK),
        gu_tab, pool_part, global_features,
        W_gn, W_gin, W_gout, bn_all, W_hn, W_he, W_hu, bh_all)
    return edges, nodes, globals_out


# speedup vs baseline: 6.3265x; 1.0328x over previous
import functools

import jax
import jax.numpy as jnp
from jax import lax
from jax.experimental import pallas as pl
from jax.experimental.pallas import tpu as pltpu
from jax.experimental.pallas import tpu_sc as plsc

N = 10000
E = 320000
G = 8
D = 128
NC = 2
NS = 16
NW = NC * NS
CA = 32
CHUNKS_A = E // CA
CPW_A = -(-CHUNKS_A // NW)
CB = 128
CHUNKS_B = E // CB
CPW_B = -(-CHUNKS_B // NW)
RPT = 632
RPT_LAST = N - (NS - 1) * RPT
EBLK = 4000
NBLK = 2000
def _dot(a, b):
    return jnp.dot(a, b, preferred_element_type=jnp.float32)


def _tables_body(nf_ref, gf_ref, wfs_ref, bfs_ref, wfr_ref, bfr_ref,
                 wfu_ref, bfu_ref, wgu_ref, bgu_ref,
                 fs_ref, fr_ref, fu_ref, gu_ref):
    nf = nf_ref[...]
    fs_ref[...] = _dot(nf, wfs_ref[...]) + bfs_ref[...]
    fr_ref[...] = _dot(nf, wfr_ref[...]) + bfr_ref[...]
    gf = gf_ref[...]
    fu_ref[...] = _dot(gf, wfu_ref[...]) + bfu_ref[...]
    gu_ref[...] = _dot(gf, wgu_ref[...]) + bgu_ref[...]


_tables_call = pl.pallas_call(
    _tables_body,
    out_shape=(
        jax.ShapeDtypeStruct((N, D), jnp.float32),
        jax.ShapeDtypeStruct((N, D), jnp.float32),
        jax.ShapeDtypeStruct((G, D), jnp.float32),
        jax.ShapeDtypeStruct((G, D), jnp.float32),
    ),
)


def _fe_body(ef_ref, gid_ref, fu_ref, wfe_ref, bfe_ref, out_ref):
    fe = _dot(ef_ref[...], wfe_ref[...]) + bfe_ref[...]
    gid = gid_ref[0, 0, :]
    onehot = (gid[:, None] == lax.broadcasted_iota(jnp.int32, (1, G), 1)
              ).astype(jnp.float32)
    out_ref[...] = fe + _dot(onehot, fu_ref[...])


_fe_call = pl.pallas_call(
    _fe_body,
    grid=(E // EBLK,),
    in_specs=[
        pl.BlockSpec((EBLK, 16), lambda i: (i, 0)),
        pl.BlockSpec((1, 1, EBLK), lambda i: (i, 0, 0)),
        pl.BlockSpec((G, D), lambda i: (0, 0)),
        pl.BlockSpec((16, D), lambda i: (0, 0)),
        pl.BlockSpec((1, D), lambda i: (0, 0)),
    ],
    out_specs=pl.BlockSpec((EBLK, D), lambda i: (i, 0)),
    out_shape=jax.ShapeDtypeStruct((E, D), jnp.float32),
)


_MESH = plsc.VectorSubcoreMesh(core_axis_name="c", subcore_axis_name="s",
                               num_cores=NC, num_subcores=NS)


def _acc_pieces(s, cb):
    start = s * RPT
    for j in range(4):
        cb(start + j * 128, 128)

    @pl.when(s < NS - 1)
    def _():
        cb(start + 512, RPT - 512)

    @pl.when(s == NS - 1)
    def _():
        cb(start + 512, RPT_LAST - 512)


def _acc_pieces_small(s, cb):
    start = s * RPT
    for j in range(16):
        cb(start + j * 32, 32)

    @pl.when(s < NS - 1)
    def _():
        for j in range(3):
            cb(start + 512 + j * 32, 32)
        cb(start + 608, RPT - 608)

    @pl.when(s == NS - 1)
    def _():
        cb(start + 512, RPT_LAST - 512)


def _zero_buf(buf, rows):
    zv = jnp.zeros((16,), jnp.float32)

    def _zrow(r, carry):
        for j in range(D // 16):
            buf[r, pl.ds(j * 16, 16)] = zv
        return carry

    lax.fori_loop(0, rows, _zrow, 0)


@functools.partial(
    pl.kernel,
    out_type=(
        jax.ShapeDtypeStruct((E, D), jnp.float32),
        jax.ShapeDtypeStruct((NC * N, D), jnp.float32),
        jax.ShapeDtypeStruct((NC * N, D), jnp.float32),
        jax.ShapeDtypeStruct((NC * NS * G, D), jnp.float32),
    ),
    mesh=_MESH,
    scratch_types=(
        [pltpu.VMEM((CA,), jnp.int32)] * 8
        + [pltpu.VMEM((CA, D), jnp.float32)] * 8
        + [pltpu.VMEM_SHARED((N, D), jnp.float32),
           pltpu.VMEM_SHARED((NS * G, D), jnp.float32)]
        + [pltpu.SemaphoreType.DMA] * 12
    ),
)
def _sc_fused_pass(fe_hbm, fs_hbm, fr_hbm, snd_hbm, rcv_hbm, gid_hbm,
                   edges_hbm, aggin_hbm, aggout_hbm, pool_hbm,
                   si0, ri0, si1, ri1, si2, ri2, si3, ri3,
                   fs0, fr0, fe0, ou0, fs1, fr1, fe1, ou1,
                   acc_sh, pool_sh,
                   smi0, smi1, smi2, smi3, smg0, smg1, smf0, smf1,
                   sms0, sms1, sma0, sma1):
    c = lax.axis_index("c")
    s = lax.axis_index("s")
    wid = s * NC + c
    isets = [(si0, ri0, smi0), (si1, ri1, smi1),
             (si2, ri2, smi2), (si3, ri3, smi3)]
    dsets = [(fs0, fr0, fe0, ou0, smg0, smf0, sms0, sma0),
             (fs1, fr1, fe1, ou1, smg1, smf1, sms1, sma1)]

    _zero_buf(fe0, CA)
    _acc_pieces_small(s, lambda off, sz: pltpu.sync_copy(
        fe0.at[pl.ds(0, sz)], acc_sh.at[pl.ds(off, sz)]))
    plsc.subcore_barrier()

    def cid_of(j):
        return wid + j * NW

    def issue_idx(j, iset):
        sidx, ridx, smi = iset

        @pl.when(cid_of(j) < CHUNKS_A)
        def _():
            base = cid_of(j) * CA
            pltpu.async_copy(snd_hbm.at[pl.ds(base, CA)], sidx, smi)
            pltpu.async_copy(rcv_hbm.at[pl.ds(base, CA)], ridx, smi)

    def prefetch_data(j, iset, dset):
        sidx, ridx, smi = iset
        fs_b, fr_b, fe_b, _, smg, smf, _, _ = dset

        @pl.when(cid_of(j) < CHUNKS_A)
        def _():
            for _ in range(2):
                pltpu.make_async_copy(
                    snd_hbm.at[pl.ds(0, CA)], sidx, smi).wait()
            pltpu.async_copy(fs_hbm.at[sidx], fs_b, smg)
            pltpu.async_copy(fr_hbm.at[ridx], fr_b, smg)
            pltpu.async_copy(fe_hbm.at[pl.ds(cid_of(j) * CA, CA)],
                             fe_b, smf)

    def step(k, icur, invt, inxt, dcur, dnxt):
        sidx, ridx, _ = icur
        fs_b, fr_b, fe_b, out_b, smg, smf, sms, sma = dcur
        valid_k = cid_of(k) < CHUNKS_A

        prefetch_data(k + 1, inxt, dnxt)
        issue_idx(k + 2, invt)

        @pl.when(valid_k)
        def _():
            pltpu.make_async_copy(fs_hbm.at[sidx], fs_b, smg).wait()
            pltpu.make_async_copy(fr_hbm.at[ridx], fr_b, smg).wait()
            pltpu.make_async_copy(
                fe_hbm.at[pl.ds(0, CA)], fe_b, smf).wait()

            def _crow(r, cy):
                for rr in range(2):
                    for j in range(D // 16):
                        sl = pl.ds(j * 16, 16)
                        v = (fe_b[2 * r + rr, sl] + fs_b[2 * r + rr, sl]
                             + fr_b[2 * r + rr, sl])
                        out_b[2 * r + rr, sl] = jnp.maximum(v, 0.0)
                return cy

            lax.fori_loop(0, CA // 2, _crow, 0)

        _, _, _, outn, _, _, smsn, sman = dnxt
        _, rin, _ = inxt

        @pl.when((k >= 1) & (cid_of(k - 1) < CHUNKS_A))
        def _():
            pltpu.make_async_copy(
                outn, edges_hbm.at[pl.ds(0, CA)], smsn).wait()
            pltpu.make_async_copy(outn, acc_sh.at[rin], sman).wait()

        @pl.when(valid_k)
        def _():
            base = cid_of(k) * CA
            pltpu.async_copy(out_b, edges_hbm.at[pl.ds(base, CA)], sms)
            pltpu.async_copy(out_b, acc_sh.at[ridx], sma, add=True)

    issue_idx(0, isets[0])
    issue_idx(1, isets[1])
    prefetch_data(0, isets[0], dsets[0])

    def _quad(t, carry):
        k = 4 * t
        step(k, isets[0], isets[2], isets[1], dsets[0], dsets[1])
        step(k + 1, isets[1], isets[3], isets[2], dsets[1], dsets[0])
        step(k + 2, isets[2], isets[0], isets[3], dsets[0], dsets[1])
        step(k + 3, isets[3], isets[1], isets[0], dsets[1], dsets[0])
        return carry

    lax.fori_loop(0, (CPW_A + 4) // 4, _quad, 0)
    kl = (CPW_A + 4) // 4 * 4 - 1
    _, _, _, outt, _, _, smst, smat = dsets[kl % 2]
    _, rit, _ = isets[kl % 4]

    @pl.when(cid_of(kl) < CHUNKS_A)
    def _():
        pltpu.make_async_copy(
            outt, edges_hbm.at[pl.ds(0, CA)], smst).wait()
        pltpu.make_async_copy(outt, acc_sh.at[rit], smat).wait()

    plsc.subcore_barrier()
    _acc_pieces(s, lambda off, sz: pltpu.sync_copy(
        acc_sh.at[pl.ds(off, sz)], aggin_hbm.at[pl.ds(c * N + off, sz)]))

    _zero_buf(fe0, CA)
    _acc_pieces_small(s, lambda off, sz: pltpu.sync_copy(
        fe0.at[pl.ds(0, sz)], acc_sh.at[pl.ds(off, sz)]))
    pltpu.sync_copy(fe0.at[pl.ds(0, G)], pool_sh.at[pl.ds(s * G, G)])
    plsc.subcore_barrier()

    bisets = [(si0, ri0, smi0), (si1, ri1, smi1), (si2, ri2, smi2)]
    bsets = [(fs0, smg0), (fr0, smg1), (fe0, smf0)]
    smas = [sma0, sma1]

    def cid_b(j):
        return c + 2 * s + j * NW

    def issue_idx_b(j, iset):
        sidx, pidx, smi = iset

        @pl.when(cid_b(j) < CHUNKS_A)
        def _():
            base = cid_b(j) * CA
            pltpu.async_copy(snd_hbm.at[pl.ds(base, CA)], sidx, smi)
            pltpu.async_copy(gid_hbm.at[pl.ds(base, CA)], pidx, smi)

    def prefetch_b(j, dset):
        ed_b, smf = dset

        @pl.when(cid_b(j) < CHUNKS_A)
        def _():
            pltpu.async_copy(
                edges_hbm.at[pl.ds(cid_b(j) * CA, CA)], ed_b, smf)

    def step_b(k, icur, invt, dcur, dnxt, sma_c, sma_n):
        sidx, pidx, smi = icur
        ed_b, smf = dcur

        prefetch_b(k + 1, dnxt)

        @pl.when(cid_b(k) < CHUNKS_A)
        def _():
            pltpu.make_async_copy(
                edges_hbm.at[pl.ds(0, CA)], ed_b, smf).wait()
            for _ in range(2):
                pltpu.make_async_copy(
                    snd_hbm.at[pl.ds(0, CA)], sidx, smi).wait()
            for i in range(CA // 16):
                sl = pl.ds(i * 16, 16)
                pidx[sl] = pidx[sl] + s * G
            pltpu.async_copy(ed_b, acc_sh.at[sidx], sma_c, add=True)
            pltpu.async_copy(ed_b, pool_sh.at[pidx], sma_c, add=True)

        @pl.when((k >= 1) & (cid_b(k - 1) < CHUNKS_A))
        def _():
            pltpu.make_async_copy(fs0, acc_sh.at[si0], sma_n).wait()
            pltpu.make_async_copy(fs0, pool_sh.at[ri0], sma_n).wait()

        issue_idx_b(k + 2, invt)

    issue_idx_b(0, bisets[0])
    issue_idx_b(1, bisets[1])
    prefetch_b(0, bsets[0])

    def _hex(t, carry):
        k = 6 * t
        for u in range(6):
            step_b(k + u, bisets[u % 3], bisets[(u + 2) % 3],
                   bsets[u % 3], bsets[(u + 1) % 3],
                   smas[u % 2], smas[(u + 1) % 2])
        return carry

    lax.fori_loop(0, (CPW_A + 6) // 6, _hex, 0)
    plsc.subcore_barrier()
    _acc_pieces(s, lambda off, sz: pltpu.sync_copy(
        acc_sh.at[pl.ds(off, sz)], aggout_hbm.at[pl.ds(c * N + off, sz)]))
    pltpu.sync_copy(pool_sh.at[pl.ds(s * G, G)],
                    pool_hbm.at[pl.ds((c * NS + s) * G, G)])


def _node_body(nf_ref, ai0, ai1, ao0, ao1, gidn_ref, gu_ref, pool_ref,
               gf_ref, wgn, wgin, wgout, bn, whn, whe, whu, bh,
               nodes_ref, glob_ref, npool_ref):
    i = pl.program_id(0)
    agg_in = ai0[...] + ai1[...]
    agg_out = ao0[...] + ao1[...]
    x = (_dot(nf_ref[...], wgn[...]) + _dot(agg_in, wgin[...])
         + _dot(agg_out, wgout[...]) + bn[...])
    gid = gidn_ref[0, 0, :]
    onehot = (gid[:, None] == lax.broadcasted_iota(jnp.int32, (1, G), 1)
              ).astype(jnp.float32)
    x = x + _dot(onehot, gu_ref[...])
    nodes = jnp.maximum(x, 0.0)
    nodes_ref[...] = nodes
    onehot_t = (lax.broadcasted_iota(jnp.int32, (G, NBLK), 0) == gid[None, :]
                ).astype(jnp.float32)
    pp = _dot(onehot_t, nodes)

    @pl.when(i == 0)
    def _():
        npool_ref[...] = pp

    @pl.when(i > 0)
    def _():
        npool_ref[...] += pp

    @pl.when(i == pl.num_programs(0) - 1)
    def _():
        ep = pool_ref[pl.ds(0, G), :]
        for j in range(1, NC * NS):
            ep = ep + pool_ref[pl.ds(j * G, G), :]
        glob_ref[...] = (_dot(ep, whe[...]) + _dot(npool_ref[...], whn[...])
                         + _dot(gf_ref[...], whu[...]) + bh[...])


_node_call = pl.pallas_call(
    _node_body,
    grid=(N // NBLK,),
    in_specs=[
        pl.BlockSpec((NBLK, D), lambda i: (i, 0)),
        pl.BlockSpec((NBLK, D), lambda i: (i, 0)),
        pl.BlockSpec((NBLK, D), lambda i: (i + N // NBLK, 0)),
        pl.BlockSpec((NBLK, D), lambda i: (i, 0)),
        pl.BlockSpec((NBLK, D), lambda i: (i + N // NBLK, 0)),
        pl.BlockSpec((1, 1, NBLK), lambda i: (i, 0, 0)),
        pl.BlockSpec((G, D), lambda i: (0, 0)),
        pl.BlockSpec((NC * NS * G, D), lambda i: (0, 0)),
        pl.BlockSpec((G, 16), lambda i: (0, 0)),
        pl.BlockSpec((D, D), lambda i: (0, 0)),
        pl.BlockSpec((D, D), lambda i: (0, 0)),
        pl.BlockSpec((D, D), lambda i: (0, 0)),
        pl.BlockSpec((1, D), lambda i: (0, 0)),
        pl.BlockSpec((D, D), lambda i: (0, 0)),
        pl.BlockSpec((D, D), lambda i: (0, 0)),
        pl.BlockSpec((16, D), lambda i: (0, 0)),
        pl.BlockSpec((1, D), lambda i: (0, 0)),
    ],
    out_specs=(
        pl.BlockSpec((NBLK, D), lambda i: (i, 0)),
        pl.BlockSpec((G, D), lambda i: (0, 0)),
    ),
    out_shape=(
        jax.ShapeDtypeStruct((N, D), jnp.float32),
        jax.ShapeDtypeStruct((G, D), jnp.float32),
    ),
    scratch_shapes=[pltpu.VMEM((G, D), jnp.float32)],
)


def kernel(node_features, edge_features, global_features, senders, receivers,
           edge_graph_ids, node_graph_ids,
           W_fe, b_fe, W_fs, b_fs, W_fr, b_fr, W_fu, b_fu,
           W_gn, b_gn, W_gin, b_gin, W_gout, b_gout, W_gu, b_gu,
           W_hn, b_hn, W_he, b_he, W_hu, b_hu):
    fs_tab, fr_tab, fu_tab, gu_tab = _tables_call(
        node_features, global_features,
        W_fs, b_fs.reshape(1, D), W_fr, b_fr.reshape(1, D),
        W_fu, b_fu.reshape(1, D), W_gu, b_gu.reshape(1, D))
    fe_all = _fe_call(edge_features,
                      edge_graph_ids.reshape(E // EBLK, 1, EBLK),
                      fu_tab, W_fe, b_fe.reshape(1, D))
    edges, aggin_part, aggout_part, pool_part = _sc_fused_pass(
        fe_all, fs_tab, fr_tab, senders, receivers, edge_graph_ids)
    bn_all = (b_gn + b_gin + b_gout).reshape(1, D)
    bh_all = (b_he + b_hn + b_hu).reshape(1, D)
    nodes, globals_out = _node_call(
        node_features, aggin_part, aggin_part, aggout_part, aggout_part,
        node_graph_ids.reshape(N // NBLK, 1, NBLK),
        gu_tab, pool_part, global_features,
        W_gn, W_gin, W_gout, bn_all, W_hn, W_he, W_hu, bh_all)
    return edges, nodes, globals_out


# speedup vs baseline: 6.3422x; 1.0025x over previous
import functools

import jax
import jax.numpy as jnp
from jax import lax
from jax.experimental import pallas as pl
from jax.experimental.pallas import tpu as pltpu
from jax.experimental.pallas import tpu_sc as plsc

N = 10000
E = 320000
G = 8
D = 128
NC = 2
NS = 16
NW = NC * NS
CA = 32
CHUNKS_A = E // CA
CPW_A = -(-CHUNKS_A // NW)
CB = 128
CHUNKS_B = E // CB
CPW_B = -(-CHUNKS_B // NW)
RPT = 632
RPT_LAST = N - (NS - 1) * RPT
EBLK = 4000
NBLK = 2000
def _dot(a, b):
    return jnp.dot(a, b, preferred_element_type=jnp.float32)


def _fe_body(ef_ref, gid_ref, nf_ref, gf_ref,
             wfe_ref, bfe_ref, wfs_ref, bfs_ref, wfr_ref, bfr_ref,
             wfu_ref, bfu_ref, wgu_ref, bgu_ref,
             out_ref, fs_ref, fr_ref, gu_ref, fu_sc):
    i = pl.program_id(0)

    @pl.when(i == 0)
    def _():
        nf = nf_ref[...]
        fs_ref[...] = _dot(nf, wfs_ref[...]) + bfs_ref[...]
        fr_ref[...] = _dot(nf, wfr_ref[...]) + bfr_ref[...]
        gf = gf_ref[...]
        fu_sc[...] = _dot(gf, wfu_ref[...]) + bfu_ref[...]
        gu_ref[...] = _dot(gf, wgu_ref[...]) + bgu_ref[...]

    fe = _dot(ef_ref[...], wfe_ref[...]) + bfe_ref[...]
    gid = gid_ref[0, 0, :]
    onehot = (gid[:, None] == lax.broadcasted_iota(jnp.int32, (1, G), 1)
              ).astype(jnp.float32)
    out_ref[...] = fe + _dot(onehot, fu_sc[...])


_fe_call = pl.pallas_call(
    _fe_body,
    grid=(E // EBLK,),
    in_specs=[
        pl.BlockSpec((EBLK, 16), lambda i: (i, 0)),
        pl.BlockSpec((1, 1, EBLK), lambda i: (i, 0, 0)),
        pl.BlockSpec((N, D), lambda i: (0, 0)),
        pl.BlockSpec((G, 16), lambda i: (0, 0)),
        pl.BlockSpec((16, D), lambda i: (0, 0)),
        pl.BlockSpec((1, D), lambda i: (0, 0)),
        pl.BlockSpec((D, D), lambda i: (0, 0)),
        pl.BlockSpec((1, D), lambda i: (0, 0)),
        pl.BlockSpec((D, D), lambda i: (0, 0)),
        pl.BlockSpec((1, D), lambda i: (0, 0)),
        pl.BlockSpec((16, D), lambda i: (0, 0)),
        pl.BlockSpec((1, D), lambda i: (0, 0)),
        pl.BlockSpec((16, D), lambda i: (0, 0)),
        pl.BlockSpec((1, D), lambda i: (0, 0)),
    ],
    out_specs=(
        pl.BlockSpec((EBLK, D), lambda i: (i, 0)),
        pl.BlockSpec((N, D), lambda i: (0, 0)),
        pl.BlockSpec((N, D), lambda i: (0, 0)),
        pl.BlockSpec((G, D), lambda i: (0, 0)),
    ),
    out_shape=(
        jax.ShapeDtypeStruct((E, D), jnp.float32),
        jax.ShapeDtypeStruct((N, D), jnp.float32),
        jax.ShapeDtypeStruct((N, D), jnp.float32),
        jax.ShapeDtypeStruct((G, D), jnp.float32),
    ),
    scratch_shapes=[pltpu.VMEM((G, D), jnp.float32)],
)


_MESH = plsc.VectorSubcoreMesh(core_axis_name="c", subcore_axis_name="s",
                               num_cores=NC, num_subcores=NS)


def _acc_pieces(s, cb):
    start = s * RPT
    for j in range(4):
        cb(start + j * 128, 128)

    @pl.when(s < NS - 1)
    def _():
        cb(start + 512, RPT - 512)

    @pl.when(s == NS - 1)
    def _():
        cb(start + 512, RPT_LAST - 512)


def _acc_pieces_small(s, cb):
    start = s * RPT
    for j in range(16):
        cb(start + j * 32, 32)

    @pl.when(s < NS - 1)
    def _():
        for j in range(3):
            cb(start + 512 + j * 32, 32)
        cb(start + 608, RPT - 608)

    @pl.when(s == NS - 1)
    def _():
        cb(start + 512, RPT_LAST - 512)


def _zero_buf(buf, rows):
    zv = jnp.zeros((16,), jnp.float32)

    def _zrow(r, carry):
        for j in range(D // 16):
            buf[r, pl.ds(j * 16, 16)] = zv
        return carry

    lax.fori_loop(0, rows, _zrow, 0)


@functools.partial(
    pl.kernel,
    out_type=(
        jax.ShapeDtypeStruct((E, D), jnp.float32),
        jax.ShapeDtypeStruct((NC * N, D), jnp.float32),
        jax.ShapeDtypeStruct((NC * N, D), jnp.float32),
        jax.ShapeDtypeStruct((NC * NS * G, D), jnp.float32),
    ),
    mesh=_MESH,
    scratch_types=(
        [pltpu.VMEM((CA,), jnp.int32)] * 8
        + [pltpu.VMEM((CA, D), jnp.float32)] * 8
        + [pltpu.VMEM_SHARED((N, D), jnp.float32),
           pltpu.VMEM_SHARED((NS * G, D), jnp.float32)]
        + [pltpu.SemaphoreType.DMA] * 12
    ),
)
def _sc_fused_pass(fe_hbm, fs_hbm, fr_hbm, snd_hbm, rcv_hbm, gid_hbm,
                   edges_hbm, aggin_hbm, aggout_hbm, pool_hbm,
                   si0, ri0, si1, ri1, si2, ri2, si3, ri3,
                   fs0, fr0, fe0, ou0, fs1, fr1, fe1, ou1,
                   acc_sh, pool_sh,
                   smi0, smi1, smi2, smi3, smg0, smg1, smf0, smf1,
                   sms0, sms1, sma0, sma1):
    c = lax.axis_index("c")
    s = lax.axis_index("s")
    wid = s * NC + c
    isets = [(si0, ri0, smi0), (si1, ri1, smi1),
             (si2, ri2, smi2), (si3, ri3, smi3)]
    dsets = [(fs0, fr0, fe0, ou0, smg0, smf0, sms0, sma0),
             (fs1, fr1, fe1, ou1, smg1, smf1, sms1, sma1)]

    _zero_buf(fe0, CA)
    _acc_pieces_small(s, lambda off, sz: pltpu.sync_copy(
        fe0.at[pl.ds(0, sz)], acc_sh.at[pl.ds(off, sz)]))
    plsc.subcore_barrier()

    def cid_of(j):
        return wid + j * NW

    def issue_idx(j, iset):
        sidx, ridx, smi = iset

        @pl.when(cid_of(j) < CHUNKS_A)
        def _():
            base = cid_of(j) * CA
            pltpu.async_copy(snd_hbm.at[pl.ds(base, CA)], sidx, smi)
            pltpu.async_copy(rcv_hbm.at[pl.ds(base, CA)], ridx, smi)

    def prefetch_data(j, iset, dset):
        sidx, ridx, smi = iset
        fs_b, fr_b, fe_b, _, smg, smf, _, _ = dset

        @pl.when(cid_of(j) < CHUNKS_A)
        def _():
            for _ in range(2):
                pltpu.make_async_copy(
                    snd_hbm.at[pl.ds(0, CA)], sidx, smi).wait()
            pltpu.async_copy(fs_hbm.at[sidx], fs_b, smg)
            pltpu.async_copy(fr_hbm.at[ridx], fr_b, smg)
            pltpu.async_copy(fe_hbm.at[pl.ds(cid_of(j) * CA, CA)],
                             fe_b, smf)

    def step(k, icur, invt, inxt, dcur, dnxt):
        sidx, ridx, _ = icur
        fs_b, fr_b, fe_b, out_b, smg, smf, sms, sma = dcur
        valid_k = cid_of(k) < CHUNKS_A

        prefetch_data(k + 1, inxt, dnxt)
        issue_idx(k + 2, invt)

        @pl.when(valid_k)
        def _():
            pltpu.make_async_copy(fs_hbm.at[sidx], fs_b, smg).wait()
            pltpu.make_async_copy(fr_hbm.at[ridx], fr_b, smg).wait()
            pltpu.make_async_copy(
                fe_hbm.at[pl.ds(0, CA)], fe_b, smf).wait()

            def _crow(r, cy):
                for rr in range(2):
                    for j in range(D // 16):
                        sl = pl.ds(j * 16, 16)
                        v = (fe_b[2 * r + rr, sl] + fs_b[2 * r + rr, sl]
                             + fr_b[2 * r + rr, sl])
                        out_b[2 * r + rr, sl] = jnp.maximum(v, 0.0)
                return cy

            lax.fori_loop(0, CA // 2, _crow, 0)

        _, _, _, outn, _, _, smsn, sman = dnxt
        _, rin, _ = inxt

        @pl.when((k >= 1) & (cid_of(k - 1) < CHUNKS_A))
        def _():
            pltpu.make_async_copy(
                outn, edges_hbm.at[pl.ds(0, CA)], smsn).wait()
            pltpu.make_async_copy(outn, acc_sh.at[rin], sman).wait()

        @pl.when(valid_k)
        def _():
            base = cid_of(k) * CA
            pltpu.async_copy(out_b, edges_hbm.at[pl.ds(base, CA)], sms)
            pltpu.async_copy(out_b, acc_sh.at[ridx], sma, add=True)

    issue_idx(0, isets[0])
    issue_idx(1, isets[1])
    prefetch_data(0, isets[0], dsets[0])

    def _quad(t, carry):
        k = 4 * t
        step(k, isets[0], isets[2], isets[1], dsets[0], dsets[1])
        step(k + 1, isets[1], isets[3], isets[2], dsets[1], dsets[0])
        step(k + 2, isets[2], isets[0], isets[3], dsets[0], dsets[1])
        step(k + 3, isets[3], isets[1], isets[0], dsets[1], dsets[0])
        return carry

    lax.fori_loop(0, (CPW_A + 4) // 4, _quad, 0)
    kl = (CPW_A + 4) // 4 * 4 - 1
    _, _, _, outt, _, _, smst, smat = dsets[kl % 2]
    _, rit, _ = isets[kl % 4]

    @pl.when(cid_of(kl) < CHUNKS_A)
    def _():
        pltpu.make_async_copy(
            outt, edges_hbm.at[pl.ds(0, CA)], smst).wait()
        pltpu.make_async_copy(outt, acc_sh.at[rit], smat).wait()

    plsc.subcore_barrier()
    _acc_pieces(s, lambda off, sz: pltpu.sync_copy(
        acc_sh.at[pl.ds(off, sz)], aggin_hbm.at[pl.ds(c * N + off, sz)]))

    _zero_buf(fe0, CA)
    _acc_pieces_small(s, lambda off, sz: pltpu.sync_copy(
        fe0.at[pl.ds(0, sz)], acc_sh.at[pl.ds(off, sz)]))
    pltpu.sync_copy(fe0.at[pl.ds(0, G)], pool_sh.at[pl.ds(s * G, G)])
    plsc.subcore_barrier()

    bisets = [(si0, ri0, smi0), (si1, ri1, smi1), (si2, ri2, smi2)]
    bsets = [(fs0, smg0), (fr0, smg1), (fe0, smf0)]
    smas = [sma0, sma1]

    def cid_b(j):
        return c + 2 * s + j * NW

    def issue_idx_b(j, iset):
        sidx, pidx, smi = iset

        @pl.when(cid_b(j) < CHUNKS_A)
        def _():
            base = cid_b(j) * CA
            pltpu.async_copy(snd_hbm.at[pl.ds(base, CA)], sidx, smi)
            pltpu.async_copy(gid_hbm.at[pl.ds(base, CA)], pidx, smi)

    def prefetch_b(j, dset):
        ed_b, smf = dset

        @pl.when(cid_b(j) < CHUNKS_A)
        def _():
            pltpu.async_copy(
                edges_hbm.at[pl.ds(cid_b(j) * CA, CA)], ed_b, smf)

    def step_b(k, icur, invt, dcur, dnxt, sma_c, sma_n):
        sidx, pidx, smi = icur
        ed_b, smf = dcur

        prefetch_b(k + 1, dnxt)

        @pl.when(cid_b(k) < CHUNKS_A)
        def _():
            pltpu.make_async_copy(
                edges_hbm.at[pl.ds(0, CA)], ed_b, smf).wait()
            for _ in range(2):
                pltpu.make_async_copy(
                    snd_hbm.at[pl.ds(0, CA)], sidx, smi).wait()
            for i in range(CA // 16):
                sl = pl.ds(i * 16, 16)
                pidx[sl] = pidx[sl] + s * G
            pltpu.async_copy(ed_b, acc_sh.at[sidx], sma_c, add=True)
            pltpu.async_copy(ed_b, pool_sh.at[pidx], sma_c, add=True)

        @pl.when((k >= 1) & (cid_b(k - 1) < CHUNKS_A))
        def _():
            pltpu.make_async_copy(fs0, acc_sh.at[si0], sma_n).wait()
            pltpu.make_async_copy(fs0, pool_sh.at[ri0], sma_n).wait()

        issue_idx_b(k + 2, invt)

    issue_idx_b(0, bisets[0])
    issue_idx_b(1, bisets[1])
    prefetch_b(0, bsets[0])

    def _hex(t, carry):
        k = 6 * t
        for u in range(6):
            step_b(k + u, bisets[u % 3], bisets[(u + 2) % 3],
                   bsets[u % 3], bsets[(u + 1) % 3],
                   smas[u % 2], smas[(u + 1) % 2])
        return carry

    lax.fori_loop(0, (CPW_A + 6) // 6, _hex, 0)
    plsc.subcore_barrier()
    _acc_pieces(s, lambda off, sz: pltpu.sync_copy(
        acc_sh.at[pl.ds(off, sz)], aggout_hbm.at[pl.ds(c * N + off, sz)]))
    pltpu.sync_copy(pool_sh.at[pl.ds(s * G, G)],
                    pool_hbm.at[pl.ds((c * NS + s) * G, G)])


def _node_body(nf_ref, ai0, ai1, ao0, ao1, gidn_ref, gu_ref, pool_ref,
               gf_ref, wgn, wgin, wgout, bn, whn, whe, whu, bh,
               nodes_ref, glob_ref, npool_ref):
    i = pl.program_id(0)
    agg_in = ai0[...] + ai1[...]
    agg_out = ao0[...] + ao1[...]
    x = (_dot(nf_ref[...], wgn[...]) + _dot(agg_in, wgin[...])
         + _dot(agg_out, wgout[...]) + bn[...])
    gid = gidn_ref[0, 0, :]
    onehot = (gid[:, None] == lax.broadcasted_iota(jnp.int32, (1, G), 1)
              ).astype(jnp.float32)
    x = x + _dot(onehot, gu_ref[...])
    nodes = jnp.maximum(x, 0.0)
    nodes_ref[...] = nodes
    onehot_t = (lax.broadcasted_iota(jnp.int32, (G, NBLK), 0) == gid[None, :]
                ).astype(jnp.float32)
    pp = _dot(onehot_t, nodes)

    @pl.when(i == 0)
    def _():
        npool_ref[...] = pp

    @pl.when(i > 0)
    def _():
        npool_ref[...] += pp

    @pl.when(i == pl.num_programs(0) - 1)
    def _():
        ep = pool_ref[pl.ds(0, G), :]
        for j in range(1, NC * NS):
            ep = ep + pool_ref[pl.ds(j * G, G), :]
        glob_ref[...] = (_dot(ep, whe[...]) + _dot(npool_ref[...], whn[...])
                         + _dot(gf_ref[...], whu[...]) + bh[...])


_node_call = pl.pallas_call(
    _node_body,
    grid=(N // NBLK,),
    in_specs=[
        pl.BlockSpec((NBLK, D), lambda i: (i, 0)),
        pl.BlockSpec((NBLK, D), lambda i: (i, 0)),
        pl.BlockSpec((NBLK, D), lambda i: (i + N // NBLK, 0)),
        pl.BlockSpec((NBLK, D), lambda i: (i, 0)),
        pl.BlockSpec((NBLK, D), lambda i: (i + N // NBLK, 0)),
        pl.BlockSpec((1, 1, NBLK), lambda i: (i, 0, 0)),
        pl.BlockSpec((G, D), lambda i: (0, 0)),
        pl.BlockSpec((NC * NS * G, D), lambda i: (0, 0)),
        pl.BlockSpec((G, 16), lambda i: (0, 0)),
        pl.BlockSpec((D, D), lambda i: (0, 0)),
        pl.BlockSpec((D, D), lambda i: (0, 0)),
        pl.BlockSpec((D, D), lambda i: (0, 0)),
        pl.BlockSpec((1, D), lambda i: (0, 0)),
        pl.BlockSpec((D, D), lambda i: (0, 0)),
        pl.BlockSpec((D, D), lambda i: (0, 0)),
        pl.BlockSpec((16, D), lambda i: (0, 0)),
        pl.BlockSpec((1, D), lambda i: (0, 0)),
    ],
    out_specs=(
        pl.BlockSpec((NBLK, D), lambda i: (i, 0)),
        pl.BlockSpec((G, D), lambda i: (0, 0)),
    ),
    out_shape=(
        jax.ShapeDtypeStruct((N, D), jnp.float32),
        jax.ShapeDtypeStruct((G, D), jnp.float32),
    ),
    scratch_shapes=[pltpu.VMEM((G, D), jnp.float32)],
)


def kernel(node_features, edge_features, global_features, senders, receivers,
           edge_graph_ids, node_graph_ids,
           W_fe, b_fe, W_fs, b_fs, W_fr, b_fr, W_fu, b_fu,
           W_gn, b_gn, W_gin, b_gin, W_gout, b_gout, W_gu, b_gu,
           W_hn, b_hn, W_he, b_he, W_hu, b_hu):
    fe_all, fs_tab, fr_tab, gu_tab = _fe_call(
        edge_features, edge_graph_ids.reshape(E // EBLK, 1, EBLK),
        node_features, global_features,
        W_fe, b_fe.reshape(1, D), W_fs, b_fs.reshape(1, D),
        W_fr, b_fr.reshape(1, D), W_fu, b_fu.reshape(1, D),
        W_gu, b_gu.reshape(1, D))
    edges, aggin_part, aggout_part, pool_part = _sc_fused_pass(
        fe_all, fs_tab, fr_tab, senders, receivers, edge_graph_ids)
    bn_all = (b_gn + b_gin + b_gout).reshape(1, D)
    bh_all = (b_he + b_hn + b_hu).reshape(1, D)
    nodes, globals_out = _node_call(
        node_features, aggin_part, aggin_part, aggout_part, aggout_part,
        node_graph_ids.reshape(N // NBLK, 1, NBLK),
        gu_tab, pool_part, global_features,
        W_gn, W_gin, W_gout, bn_all, W_hn, W_he, W_hu, bh_all)
    return edges, nodes, globals_out


# speedup vs baseline: 6.5553x; 1.0336x over previous
import functools

import jax
import jax.numpy as jnp
from jax import lax
from jax.experimental import pallas as pl
from jax.experimental.pallas import tpu as pltpu
from jax.experimental.pallas import tpu_sc as plsc

N = 10000
E = 320000
G = 8
D = 128
NC = 2
NS = 16
NW = NC * NS
CA = 32
CHUNKS_A = E // CA
CPW_A = -(-CHUNKS_A // NW)
CB = 128
CHUNKS_B = E // CB
CPW_B = -(-CHUNKS_B // NW)
RPT = 632
RPT_LAST = N - (NS - 1) * RPT
EBLK = 8000
NBLK = 2000
def _dot(a, b):
    return jnp.dot(a, b, preferred_element_type=jnp.float32)


def _fe_body(ef_ref, gid_ref, nf_ref, gf_ref,
             wfe_ref, bfe_ref, wfs_ref, bfs_ref, wfr_ref, bfr_ref,
             wfu_ref, bfu_ref, wgu_ref, bgu_ref,
             out_ref, fs_ref, fr_ref, gu_ref, fu_sc):
    i = pl.program_id(0)

    @pl.when(i == 0)
    def _():
        nf = nf_ref[...]
        fs_ref[...] = _dot(nf, wfs_ref[...]) + bfs_ref[...]
        fr_ref[...] = _dot(nf, wfr_ref[...]) + bfr_ref[...]
        gf = gf_ref[...]
        fu_sc[...] = _dot(gf, wfu_ref[...]) + bfu_ref[...]
        gu_ref[...] = _dot(gf, wgu_ref[...]) + bgu_ref[...]

    fe = _dot(ef_ref[...], wfe_ref[...]) + bfe_ref[...]
    gid = gid_ref[0, 0, :]
    onehot = (gid[:, None] == lax.broadcasted_iota(jnp.int32, (1, G), 1)
              ).astype(jnp.float32)
    out_ref[...] = fe + _dot(onehot, fu_sc[...])


_fe_call = pl.pallas_call(
    _fe_body,
    grid=(E // EBLK,),
    in_specs=[
        pl.BlockSpec((EBLK, 16), lambda i: (i, 0)),
        pl.BlockSpec((1, 1, EBLK), lambda i: (i, 0, 0)),
        pl.BlockSpec((N, D), lambda i: (0, 0)),
        pl.BlockSpec((G, 16), lambda i: (0, 0)),
        pl.BlockSpec((16, D), lambda i: (0, 0)),
        pl.BlockSpec((1, D), lambda i: (0, 0)),
        pl.BlockSpec((D, D), lambda i: (0, 0)),
        pl.BlockSpec((1, D), lambda i: (0, 0)),
        pl.BlockSpec((D, D), lambda i: (0, 0)),
        pl.BlockSpec((1, D), lambda i: (0, 0)),
        pl.BlockSpec((16, D), lambda i: (0, 0)),
        pl.BlockSpec((1, D), lambda i: (0, 0)),
        pl.BlockSpec((16, D), lambda i: (0, 0)),
        pl.BlockSpec((1, D), lambda i: (0, 0)),
    ],
    out_specs=(
        pl.BlockSpec((EBLK, D), lambda i: (i, 0)),
        pl.BlockSpec((N, D), lambda i: (0, 0)),
        pl.BlockSpec((N, D), lambda i: (0, 0)),
        pl.BlockSpec((G, D), lambda i: (0, 0)),
    ),
    out_shape=(
        jax.ShapeDtypeStruct((E, D), jnp.float32),
        jax.ShapeDtypeStruct((N, D), jnp.float32),
        jax.ShapeDtypeStruct((N, D), jnp.float32),
        jax.ShapeDtypeStruct((G, D), jnp.float32),
    ),
    scratch_shapes=[pltpu.VMEM((G, D), jnp.float32)],
)


_MESH = plsc.VectorSubcoreMesh(core_axis_name="c", subcore_axis_name="s",
                               num_cores=NC, num_subcores=NS)


def _acc_pieces(s, cb):
    start = s * RPT
    for j in range(4):
        cb(start + j * 128, 128)

    @pl.when(s < NS - 1)
    def _():
        cb(start + 512, RPT - 512)

    @pl.when(s == NS - 1)
    def _():
        cb(start + 512, RPT_LAST - 512)


def _acc_pieces_small(s, cb):
    start = s * RPT
    for j in range(16):
        cb(start + j * 32, 32)

    @pl.when(s < NS - 1)
    def _():
        for j in range(3):
            cb(start + 512 + j * 32, 32)
        cb(start + 608, RPT - 608)

    @pl.when(s == NS - 1)
    def _():
        cb(start + 512, RPT_LAST - 512)


def _zero_buf(buf, rows):
    zv = jnp.zeros((16,), jnp.float32)

    def _zrow(r, carry):
        for j in range(D // 16):
            buf[r, pl.ds(j * 16, 16)] = zv
        return carry

    lax.fori_loop(0, rows, _zrow, 0)


@functools.partial(
    pl.kernel,
    out_type=(
        jax.ShapeDtypeStruct((E, D), jnp.float32),
        jax.ShapeDtypeStruct((NC * N, D), jnp.float32),
        jax.ShapeDtypeStruct((NC * N, D), jnp.float32),
        jax.ShapeDtypeStruct((NC * NS * G, D), jnp.float32),
    ),
    mesh=_MESH,
    scratch_types=(
        [pltpu.VMEM((CA,), jnp.int32)] * 8
        + [pltpu.VMEM((CA, D), jnp.float32)] * 8
        + [pltpu.VMEM_SHARED((N, D), jnp.float32),
           pltpu.VMEM_SHARED((NS * G, D), jnp.float32)]
        + [pltpu.SemaphoreType.DMA] * 12
    ),
)
def _sc_fused_pass(fe_hbm, fs_hbm, fr_hbm, snd_hbm, rcv_hbm, gid_hbm,
                   edges_hbm, aggin_hbm, aggout_hbm, pool_hbm,
                   si0, ri0, si1, ri1, si2, ri2, si3, ri3,
                   fs0, fr0, fe0, ou0, fs1, fr1, fe1, ou1,
                   acc_sh, pool_sh,
                   smi0, smi1, smi2, smi3, smg0, smg1, smf0, smf1,
                   sms0, sms1, sma0, sma1):
    c = lax.axis_index("c")
    s = lax.axis_index("s")
    wid = s * NC + c
    isets = [(si0, ri0, smi0), (si1, ri1, smi1),
             (si2, ri2, smi2), (si3, ri3, smi3)]
    dsets = [(fs0, fr0, fe0, ou0, smg0, smf0, sms0, sma0),
             (fs1, fr1, fe1, ou1, smg1, smf1, sms1, sma1)]

    _zero_buf(fe0, CA)
    _acc_pieces_small(s, lambda off, sz: pltpu.sync_copy(
        fe0.at[pl.ds(0, sz)], acc_sh.at[pl.ds(off, sz)]))
    plsc.subcore_barrier()

    def cid_of(j):
        return wid + j * NW

    def issue_idx(j, iset):
        sidx, ridx, smi = iset

        @pl.when(cid_of(j) < CHUNKS_A)
        def _():
            base = cid_of(j) * CA
            pltpu.async_copy(snd_hbm.at[pl.ds(base, CA)], sidx, smi)
            pltpu.async_copy(rcv_hbm.at[pl.ds(base, CA)], ridx, smi)

    def prefetch_data(j, iset, dset):
        sidx, ridx, smi = iset
        fs_b, fr_b, fe_b, _, smg, smf, _, _ = dset

        @pl.when(cid_of(j) < CHUNKS_A)
        def _():
            for _ in range(2):
                pltpu.make_async_copy(
                    snd_hbm.at[pl.ds(0, CA)], sidx, smi).wait()
            pltpu.async_copy(fs_hbm.at[sidx], fs_b, smg)
            pltpu.async_copy(fr_hbm.at[ridx], fr_b, smg)
            pltpu.async_copy(fe_hbm.at[pl.ds(cid_of(j) * CA, CA)],
                             fe_b, smf)

    def step(k, icur, invt, inxt, dcur, dnxt):
        sidx, ridx, _ = icur
        fs_b, fr_b, fe_b, out_b, smg, smf, sms, sma = dcur
        valid_k = cid_of(k) < CHUNKS_A

        prefetch_data(k + 1, inxt, dnxt)
        issue_idx(k + 2, invt)

        @pl.when(valid_k)
        def _():
            pltpu.make_async_copy(fs_hbm.at[sidx], fs_b, smg).wait()
            pltpu.make_async_copy(fr_hbm.at[ridx], fr_b, smg).wait()
            pltpu.make_async_copy(
                fe_hbm.at[pl.ds(0, CA)], fe_b, smf).wait()

            def _crow(r, cy):
                for rr in range(2):
                    for j in range(D // 16):
                        sl = pl.ds(j * 16, 16)
                        v = (fe_b[2 * r + rr, sl] + fs_b[2 * r + rr, sl]
                             + fr_b[2 * r + rr, sl])
                        out_b[2 * r + rr, sl] = jnp.maximum(v, 0.0)
                return cy

            lax.fori_loop(0, CA // 2, _crow, 0)

        _, _, _, outn, _, _, smsn, sman = dnxt
        _, rin, _ = inxt

        @pl.when((k >= 1) & (cid_of(k - 1) < CHUNKS_A))
        def _():
            pltpu.make_async_copy(
                outn, edges_hbm.at[pl.ds(0, CA)], smsn).wait()
            pltpu.make_async_copy(outn, acc_sh.at[rin], sman).wait()

        @pl.when(valid_k)
        def _():
            base = cid_of(k) * CA
            pltpu.async_copy(out_b, edges_hbm.at[pl.ds(base, CA)], sms)
            pltpu.async_copy(out_b, acc_sh.at[ridx], sma, add=True)

    issue_idx(0, isets[0])
    issue_idx(1, isets[1])
    prefetch_data(0, isets[0], dsets[0])

    def _quad(t, carry):
        k = 4 * t
        step(k, isets[0], isets[2], isets[1], dsets[0], dsets[1])
        step(k + 1, isets[1], isets[3], isets[2], dsets[1], dsets[0])
        step(k + 2, isets[2], isets[0], isets[3], dsets[0], dsets[1])
        step(k + 3, isets[3], isets[1], isets[0], dsets[1], dsets[0])
        return carry

    lax.fori_loop(0, (CPW_A + 4) // 4, _quad, 0)
    kl = (CPW_A + 4) // 4 * 4 - 1
    _, _, _, outt, _, _, smst, smat = dsets[kl % 2]
    _, rit, _ = isets[kl % 4]

    @pl.when(cid_of(kl) < CHUNKS_A)
    def _():
        pltpu.make_async_copy(
            outt, edges_hbm.at[pl.ds(0, CA)], smst).wait()
        pltpu.make_async_copy(outt, acc_sh.at[rit], smat).wait()

    plsc.subcore_barrier()
    _acc_pieces(s, lambda off, sz: pltpu.sync_copy(
        acc_sh.at[pl.ds(off, sz)], aggin_hbm.at[pl.ds(c * N + off, sz)]))

    _zero_buf(fe0, CA)
    _acc_pieces_small(s, lambda off, sz: pltpu.sync_copy(
        fe0.at[pl.ds(0, sz)], acc_sh.at[pl.ds(off, sz)]))
    pltpu.sync_copy(fe0.at[pl.ds(0, G)], pool_sh.at[pl.ds(s * G, G)])
    plsc.subcore_barrier()

    bisets = [(si0, ri0, smi0), (si1, ri1, smi1), (si2, ri2, smi2)]
    bsets = [(fs0, smg0), (fr0, smg1), (fe0, smf0)]
    smas = [sma0, sma1]

    def cid_b(j):
        return c + 2 * s + j * NW

    def issue_idx_b(j, iset):
        sidx, pidx, smi = iset

        @pl.when(cid_b(j) < CHUNKS_A)
        def _():
            base = cid_b(j) * CA
            pltpu.async_copy(snd_hbm.at[pl.ds(base, CA)], sidx, smi)
            pltpu.async_copy(gid_hbm.at[pl.ds(base, CA)], pidx, smi)

    def prefetch_b(j, dset):
        ed_b, smf = dset

        @pl.when(cid_b(j) < CHUNKS_A)
        def _():
            pltpu.async_copy(
                edges_hbm.at[pl.ds(cid_b(j) * CA, CA)], ed_b, smf)

    def step_b(k, icur, invt, dcur, dnxt, sma_c, sma_n):
        sidx, pidx, smi = icur
        ed_b, smf = dcur

        prefetch_b(k + 1, dnxt)

        @pl.when(cid_b(k) < CHUNKS_A)
        def _():
            pltpu.make_async_copy(
                edges_hbm.at[pl.ds(0, CA)], ed_b, smf).wait()
            for _ in range(2):
                pltpu.make_async_copy(
                    snd_hbm.at[pl.ds(0, CA)], sidx, smi).wait()
            for i in range(CA // 16):
                sl = pl.ds(i * 16, 16)
                pidx[sl] = pidx[sl] + s * G
            pltpu.async_copy(ed_b, acc_sh.at[sidx], sma_c, add=True)
            pltpu.async_copy(ed_b, pool_sh.at[pidx], sma_c, add=True)

        @pl.when((k >= 1) & (cid_b(k - 1) < CHUNKS_A))
        def _():
            pltpu.make_async_copy(fs0, acc_sh.at[si0], sma_n).wait()
            pltpu.make_async_copy(fs0, pool_sh.at[ri0], sma_n).wait()

        issue_idx_b(k + 2, invt)

    issue_idx_b(0, bisets[0])
    issue_idx_b(1, bisets[1])
    prefetch_b(0, bsets[0])

    def _hex(t, carry):
        k = 6 * t
        for u in range(6):
            step_b(k + u, bisets[u % 3], bisets[(u + 2) % 3],
                   bsets[u % 3], bsets[(u + 1) % 3],
                   smas[u % 2], smas[(u + 1) % 2])
        return carry

    lax.fori_loop(0, (CPW_A + 6) // 6, _hex, 0)
    plsc.subcore_barrier()
    _acc_pieces(s, lambda off, sz: pltpu.sync_copy(
        acc_sh.at[pl.ds(off, sz)], aggout_hbm.at[pl.ds(c * N + off, sz)]))
    pltpu.sync_copy(pool_sh.at[pl.ds(s * G, G)],
                    pool_hbm.at[pl.ds((c * NS + s) * G, G)])


def _node_body(nf_ref, ai0, ai1, ao0, ao1, gidn_ref, gu_ref, pool_ref,
               gf_ref, wgn, wgin, wgout, bn, whn, whe, whu, bh,
               nodes_ref, glob_ref, npool_ref):
    i = pl.program_id(0)
    agg_in = ai0[...] + ai1[...]
    agg_out = ao0[...] + ao1[...]
    x = (_dot(nf_ref[...], wgn[...]) + _dot(agg_in, wgin[...])
         + _dot(agg_out, wgout[...]) + bn[...])
    gid = gidn_ref[0, 0, :]
    onehot = (gid[:, None] == lax.broadcasted_iota(jnp.int32, (1, G), 1)
              ).astype(jnp.float32)
    x = x + _dot(onehot, gu_ref[...])
    nodes = jnp.maximum(x, 0.0)
    nodes_ref[...] = nodes
    onehot_t = (lax.broadcasted_iota(jnp.int32, (G, NBLK), 0) == gid[None, :]
                ).astype(jnp.float32)
    pp = _dot(onehot_t, nodes)

    @pl.when(i == 0)
    def _():
        npool_ref[...] = pp

    @pl.when(i > 0)
    def _():
        npool_ref[...] += pp

    @pl.when(i == pl.num_programs(0) - 1)
    def _():
        ep = pool_ref[pl.ds(0, G), :]
        for j in range(1, NC * NS):
            ep = ep + pool_ref[pl.ds(j * G, G), :]
        glob_ref[...] = (_dot(ep, whe[...]) + _dot(npool_ref[...], whn[...])
                         + _dot(gf_ref[...], whu[...]) + bh[...])


_node_call = pl.pallas_call(
    _node_body,
    grid=(N // NBLK,),
    in_specs=[
        pl.BlockSpec((NBLK, D), lambda i: (i, 0)),
        pl.BlockSpec((NBLK, D), lambda i: (i, 0)),
        pl.BlockSpec((NBLK, D), lambda i: (i + N // NBLK, 0)),
        pl.BlockSpec((NBLK, D), lambda i: (i, 0)),
        pl.BlockSpec((NBLK, D), lambda i: (i + N // NBLK, 0)),
        pl.BlockSpec((1, 1, NBLK), lambda i: (i, 0, 0)),
        pl.BlockSpec((G, D), lambda i: (0, 0)),
        pl.BlockSpec((NC * NS * G, D), lambda i: (0, 0)),
        pl.BlockSpec((G, 16), lambda i: (0, 0)),
        pl.BlockSpec((D, D), lambda i: (0, 0)),
        pl.BlockSpec((D, D), lambda i: (0, 0)),
        pl.BlockSpec((D, D), lambda i: (0, 0)),
        pl.BlockSpec((1, D), lambda i: (0, 0)),
        pl.BlockSpec((D, D), lambda i: (0, 0)),
        pl.BlockSpec((D, D), lambda i: (0, 0)),
        pl.BlockSpec((16, D), lambda i: (0, 0)),
        pl.BlockSpec((1, D), lambda i: (0, 0)),
    ],
    out_specs=(
        pl.BlockSpec((NBLK, D), lambda i: (i, 0)),
        pl.BlockSpec((G, D), lambda i: (0, 0)),
    ),
    out_shape=(
        jax.ShapeDtypeStruct((N, D), jnp.float32),
        jax.ShapeDtypeStruct((G, D), jnp.float32),
    ),
    scratch_shapes=[pltpu.VMEM((G, D), jnp.float32)],
)


def kernel(node_features, edge_features, global_features, senders, receivers,
           edge_graph_ids, node_graph_ids,
           W_fe, b_fe, W_fs, b_fs, W_fr, b_fr, W_fu, b_fu,
           W_gn, b_gn, W_gin, b_gin, W_gout, b_gout, W_gu, b_gu,
           W_hn, b_hn, W_he, b_he, W_hu, b_hu):
    fe_all, fs_tab, fr_tab, gu_tab = _fe_call(
        edge_features, edge_graph_ids.reshape(E // EBLK, 1, EBLK),
        node_features, global_features,
        W_fe, b_fe.reshape(1, D), W_fs, b_fs.reshape(1, D),
        W_fr, b_fr.reshape(1, D), W_fu, b_fu.reshape(1, D),
        W_gu, b_gu.reshape(1, D))
    edges, aggin_part, aggout_part, pool_part = _sc_fused_pass(
        fe_all, fs_tab, fr_tab, senders, receivers, edge_graph_ids)
    bn_all = (b_gn + b_gin + b_gout).reshape(1, D)
    bh_all = (b_he + b_hn + b_hu).reshape(1, D)
    nodes, globals_out = _node_call(
        node_features, aggin_part, aggin_part, aggout_part, aggout_part,
        node_graph_ids.reshape(N // NBLK, 1, NBLK),
        gu_tab, pool_part, global_features,
        W_gn, W_gin, W_gout, bn_all, W_hn, W_he, W_hu, bh_all)
    return edges, nodes, globals_out


# speedup vs baseline: 6.5965x; 1.0063x over previous
import functools

import jax
import jax.numpy as jnp
from jax import lax
from jax.experimental import pallas as pl
from jax.experimental.pallas import tpu as pltpu
from jax.experimental.pallas import tpu_sc as plsc

N = 10000
E = 320000
G = 8
D = 128
NC = 2
NS = 16
NW = NC * NS
CA = 32
CHUNKS_A = E // CA
CPW_A = -(-CHUNKS_A // NW)
CB = 128
CHUNKS_B = E // CB
CPW_B = -(-CHUNKS_B // NW)
RPT = 632
RPT_LAST = N - (NS - 1) * RPT
EBLK = 16000
NBLK = 2000
def _dot(a, b):
    return jnp.dot(a, b, preferred_element_type=jnp.float32)


def _fe_body(ef_ref, gid_ref, nf_ref, gf_ref,
             wfe_ref, bfe_ref, wfs_ref, bfs_ref, wfr_ref, bfr_ref,
             wfu_ref, bfu_ref, wgu_ref, bgu_ref,
             out_ref, fs_ref, fr_ref, gu_ref, fu_sc):
    i = pl.program_id(0)

    @pl.when(i == 0)
    def _():
        nf = nf_ref[...]
        fs_ref[...] = _dot(nf, wfs_ref[...]) + bfs_ref[...]
        fr_ref[...] = _dot(nf, wfr_ref[...]) + bfr_ref[...]
        gf = gf_ref[...]
        fu_sc[...] = _dot(gf, wfu_ref[...]) + bfu_ref[...]
        gu_ref[...] = _dot(gf, wgu_ref[...]) + bgu_ref[...]

    fe = _dot(ef_ref[...], wfe_ref[...]) + bfe_ref[...]
    gid = gid_ref[0, 0, :]
    onehot = (gid[:, None] == lax.broadcasted_iota(jnp.int32, (1, G), 1)
              ).astype(jnp.float32)
    out_ref[...] = fe + _dot(onehot, fu_sc[...])


_fe_call = pl.pallas_call(
    _fe_body,
    grid=(E // EBLK,),
    in_specs=[
        pl.BlockSpec((EBLK, 16), lambda i: (i, 0)),
        pl.BlockSpec((1, 1, EBLK), lambda i: (i, 0, 0)),
        pl.BlockSpec((N, D), lambda i: (0, 0)),
        pl.BlockSpec((G, 16), lambda i: (0, 0)),
        pl.BlockSpec((16, D), lambda i: (0, 0)),
        pl.BlockSpec((1, D), lambda i: (0, 0)),
        pl.BlockSpec((D, D), lambda i: (0, 0)),
        pl.BlockSpec((1, D), lambda i: (0, 0)),
        pl.BlockSpec((D, D), lambda i: (0, 0)),
        pl.BlockSpec((1, D), lambda i: (0, 0)),
        pl.BlockSpec((16, D), lambda i: (0, 0)),
        pl.BlockSpec((1, D), lambda i: (0, 0)),
        pl.BlockSpec((16, D), lambda i: (0, 0)),
        pl.BlockSpec((1, D), lambda i: (0, 0)),
    ],
    out_specs=(
        pl.BlockSpec((EBLK, D), lambda i: (i, 0)),
        pl.BlockSpec((N, D), lambda i: (0, 0)),
        pl.BlockSpec((N, D), lambda i: (0, 0)),
        pl.BlockSpec((G, D), lambda i: (0, 0)),
    ),
    out_shape=(
        jax.ShapeDtypeStruct((E, D), jnp.float32),
        jax.ShapeDtypeStruct((N, D), jnp.float32),
        jax.ShapeDtypeStruct((N, D), jnp.float32),
        jax.ShapeDtypeStruct((G, D), jnp.float32),
    ),
    scratch_shapes=[pltpu.VMEM((G, D), jnp.float32)],
)


_MESH = plsc.VectorSubcoreMesh(core_axis_name="c", subcore_axis_name="s",
                               num_cores=NC, num_subcores=NS)


def _acc_pieces(s, cb):
    start = s * RPT
    for j in range(4):
        cb(start + j * 128, 128)

    @pl.when(s < NS - 1)
    def _():
        cb(start + 512, RPT - 512)

    @pl.when(s == NS - 1)
    def _():
        cb(start + 512, RPT_LAST - 512)


def _acc_pieces_small(s, cb):
    start = s * RPT
    for j in range(16):
        cb(start + j * 32, 32)

    @pl.when(s < NS - 1)
    def _():
        for j in range(3):
            cb(start + 512 + j * 32, 32)
        cb(start + 608, RPT - 608)

    @pl.when(s == NS - 1)
    def _():
        cb(start + 512, RPT_LAST - 512)


def _zero_buf(buf, rows):
    zv = jnp.zeros((16,), jnp.float32)

    def _zrow(r, carry):
        for j in range(D // 16):
            buf[r, pl.ds(j * 16, 16)] = zv
        return carry

    lax.fori_loop(0, rows, _zrow, 0)


@functools.partial(
    pl.kernel,
    out_type=(
        jax.ShapeDtypeStruct((E, D), jnp.float32),
        jax.ShapeDtypeStruct((NC * N, D), jnp.float32),
        jax.ShapeDtypeStruct((NC * N, D), jnp.float32),
        jax.ShapeDtypeStruct((NC * NS * G, D), jnp.float32),
    ),
    mesh=_MESH,
    scratch_types=(
        [pltpu.VMEM((CA,), jnp.int32)] * 8
        + [pltpu.VMEM((CA, D), jnp.float32)] * 8
        + [pltpu.VMEM_SHARED((N, D), jnp.float32),
           pltpu.VMEM_SHARED((NS * G, D), jnp.float32)]
        + [pltpu.SemaphoreType.DMA] * 12
    ),
)
def _sc_fused_pass(fe_hbm, fs_hbm, fr_hbm, snd_hbm, rcv_hbm, gid_hbm,
                   edges_hbm, aggin_hbm, aggout_hbm, pool_hbm,
                   si0, ri0, si1, ri1, si2, ri2, si3, ri3,
                   fs0, fr0, fe0, ou0, fs1, fr1, fe1, ou1,
                   acc_sh, pool_sh,
                   smi0, smi1, smi2, smi3, smg0, smg1, smf0, smf1,
                   sms0, sms1, sma0, sma1):
    c = lax.axis_index("c")
    s = lax.axis_index("s")
    wid = s * NC + c
    isets = [(si0, ri0, smi0), (si1, ri1, smi1),
             (si2, ri2, smi2), (si3, ri3, smi3)]
    dsets = [(fs0, fr0, fe0, ou0, smg0, smf0, sms0, sma0),
             (fs1, fr1, fe1, ou1, smg1, smf1, sms1, sma1)]

    _zero_buf(fe0, CA)
    _acc_pieces_small(s, lambda off, sz: pltpu.sync_copy(
        fe0.at[pl.ds(0, sz)], acc_sh.at[pl.ds(off, sz)]))
    plsc.subcore_barrier()

    def cid_of(j):
        return wid + j * NW

    def issue_idx(j, iset):
        sidx, ridx, smi = iset

        @pl.when(cid_of(j) < CHUNKS_A)
        def _():
            base = cid_of(j) * CA
            pltpu.async_copy(snd_hbm.at[pl.ds(base, CA)], sidx, smi)
            pltpu.async_copy(rcv_hbm.at[pl.ds(base, CA)], ridx, smi)

    def prefetch_data(j, iset, dset):
        sidx, ridx, smi = iset
        fs_b, fr_b, fe_b, _, smg, smf, _, _ = dset

        @pl.when(cid_of(j) < CHUNKS_A)
        def _():
            for _ in range(2):
                pltpu.make_async_copy(
                    snd_hbm.at[pl.ds(0, CA)], sidx, smi).wait()
            pltpu.async_copy(fs_hbm.at[sidx], fs_b, smg)
            pltpu.async_copy(fr_hbm.at[ridx], fr_b, smg)
            pltpu.async_copy(fe_hbm.at[pl.ds(cid_of(j) * CA, CA)],
                             fe_b, smf)

    def step(k, icur, invt, inxt, dcur, dnxt):
        sidx, ridx, _ = icur
        fs_b, fr_b, fe_b, out_b, smg, smf, sms, sma = dcur
        valid_k = cid_of(k) < CHUNKS_A

        prefetch_data(k + 1, inxt, dnxt)
        issue_idx(k + 2, invt)

        @pl.when(valid_k)
        def _():
            pltpu.make_async_copy(fs_hbm.at[sidx], fs_b, smg).wait()
            pltpu.make_async_copy(fr_hbm.at[ridx], fr_b, smg).wait()
            pltpu.make_async_copy(
                fe_hbm.at[pl.ds(0, CA)], fe_b, smf).wait()

            def _crow(r, cy):
                for rr in range(2):
                    for j in range(D // 16):
                        sl = pl.ds(j * 16, 16)
                        v = (fe_b[2 * r + rr, sl] + fs_b[2 * r + rr, sl]
                             + fr_b[2 * r + rr, sl])
                        out_b[2 * r + rr, sl] = jnp.maximum(v, 0.0)
                return cy

            lax.fori_loop(0, CA // 2, _crow, 0)

        _, _, _, outn, _, _, smsn, sman = dnxt
        _, rin, _ = inxt

        @pl.when((k >= 1) & (cid_of(k - 1) < CHUNKS_A))
        def _():
            pltpu.make_async_copy(
                outn, edges_hbm.at[pl.ds(0, CA)], smsn).wait()
            pltpu.make_async_copy(outn, acc_sh.at[rin], sman).wait()

        @pl.when(valid_k)
        def _():
            base = cid_of(k) * CA
            pltpu.async_copy(out_b, edges_hbm.at[pl.ds(base, CA)], sms)
            pltpu.async_copy(out_b, acc_sh.at[ridx], sma, add=True)

    issue_idx(0, isets[0])
    issue_idx(1, isets[1])
    prefetch_data(0, isets[0], dsets[0])

    def _quad(t, carry):
        k = 4 * t
        step(k, isets[0], isets[2], isets[1], dsets[0], dsets[1])
        step(k + 1, isets[1], isets[3], isets[2], dsets[1], dsets[0])
        step(k + 2, isets[2], isets[0], isets[3], dsets[0], dsets[1])
        step(k + 3, isets[3], isets[1], isets[0], dsets[1], dsets[0])
        return carry

    lax.fori_loop(0, (CPW_A + 4) // 4, _quad, 0)
    kl = (CPW_A + 4) // 4 * 4 - 1
    _, _, _, outt, _, _, smst, smat = dsets[kl % 2]
    _, rit, _ = isets[kl % 4]

    @pl.when(cid_of(kl) < CHUNKS_A)
    def _():
        pltpu.make_async_copy(
            outt, edges_hbm.at[pl.ds(0, CA)], smst).wait()
        pltpu.make_async_copy(outt, acc_sh.at[rit], smat).wait()

    plsc.subcore_barrier()
    _acc_pieces(s, lambda off, sz: pltpu.sync_copy(
        acc_sh.at[pl.ds(off, sz)], aggin_hbm.at[pl.ds(c * N + off, sz)]))

    _zero_buf(fe0, CA)
    _acc_pieces_small(s, lambda off, sz: pltpu.sync_copy(
        fe0.at[pl.ds(0, sz)], acc_sh.at[pl.ds(off, sz)]))
    pltpu.sync_copy(fe0.at[pl.ds(0, G)], pool_sh.at[pl.ds(s * G, G)])
    plsc.subcore_barrier()

    bisets = [(si0, ri0, smi0), (si1, ri1, smi1), (si2, ri2, smi2)]
    bsets = [(fs0, smg0), (fr0, smg1), (fe0, smf0)]
    smas = [sma0, sma1]

    def cid_b(j):
        return c + 2 * s + j * NW

    def issue_idx_b(j, iset):
        sidx, pidx, smi = iset

        @pl.when(cid_b(j) < CHUNKS_A)
        def _():
            base = cid_b(j) * CA
            pltpu.async_copy(snd_hbm.at[pl.ds(base, CA)], sidx, smi)
            pltpu.async_copy(gid_hbm.at[pl.ds(base, CA)], pidx, smi)

    def prefetch_b(j, dset):
        ed_b, smf = dset

        @pl.when(cid_b(j) < CHUNKS_A)
        def _():
            pltpu.async_copy(
                edges_hbm.at[pl.ds(cid_b(j) * CA, CA)], ed_b, smf)

    def step_b(k, icur, invt, dcur, dnxt, sma_c, sma_n):
        sidx, pidx, smi = icur
        ed_b, smf = dcur

        prefetch_b(k + 1, dnxt)

        @pl.when(cid_b(k) < CHUNKS_A)
        def _():
            pltpu.make_async_copy(
                edges_hbm.at[pl.ds(0, CA)], ed_b, smf).wait()
            for _ in range(2):
                pltpu.make_async_copy(
                    snd_hbm.at[pl.ds(0, CA)], sidx, smi).wait()
            for i in range(CA // 16):
                sl = pl.ds(i * 16, 16)
                pidx[sl] = pidx[sl] + s * G
            pltpu.async_copy(ed_b, acc_sh.at[sidx], sma_c, add=True)
            pltpu.async_copy(ed_b, pool_sh.at[pidx], sma_c, add=True)

        @pl.when((k >= 1) & (cid_b(k - 1) < CHUNKS_A))
        def _():
            pltpu.make_async_copy(fs0, acc_sh.at[si0], sma_n).wait()
            pltpu.make_async_copy(fs0, pool_sh.at[ri0], sma_n).wait()

        issue_idx_b(k + 2, invt)

    issue_idx_b(0, bisets[0])
    issue_idx_b(1, bisets[1])
    prefetch_b(0, bsets[0])

    def _hex(t, carry):
        k = 6 * t
        for u in range(6):
            step_b(k + u, bisets[u % 3], bisets[(u + 2) % 3],
                   bsets[u % 3], bsets[(u + 1) % 3],
                   smas[u % 2], smas[(u + 1) % 2])
        return carry

    lax.fori_loop(0, (CPW_A + 6) // 6, _hex, 0)
    plsc.subcore_barrier()
    _acc_pieces(s, lambda off, sz: pltpu.sync_copy(
        acc_sh.at[pl.ds(off, sz)], aggout_hbm.at[pl.ds(c * N + off, sz)]))
    pltpu.sync_copy(pool_sh.at[pl.ds(s * G, G)],
                    pool_hbm.at[pl.ds((c * NS + s) * G, G)])


def _node_body(nf_ref, ai0, ai1, ao0, ao1, gidn_ref, gu_ref, pool_ref,
               gf_ref, wgn, wgin, wgout, bn, whn, whe, whu, bh,
               nodes_ref, glob_ref, npool_ref):
    i = pl.program_id(0)
    agg_in = ai0[...] + ai1[...]
    agg_out = ao0[...] + ao1[...]
    x = (_dot(nf_ref[...], wgn[...]) + _dot(agg_in, wgin[...])
         + _dot(agg_out, wgout[...]) + bn[...])
    gid = gidn_ref[0, 0, :]
    onehot = (gid[:, None] == lax.broadcasted_iota(jnp.int32, (1, G), 1)
              ).astype(jnp.float32)
    x = x + _dot(onehot, gu_ref[...])
    nodes = jnp.maximum(x, 0.0)
    nodes_ref[...] = nodes
    onehot_t = (lax.broadcasted_iota(jnp.int32, (G, NBLK), 0) == gid[None, :]
                ).astype(jnp.float32)
    pp = _dot(onehot_t, nodes)

    @pl.when(i == 0)
    def _():
        npool_ref[...] = pp

    @pl.when(i > 0)
    def _():
        npool_ref[...] += pp

    @pl.when(i == pl.num_programs(0) - 1)
    def _():
        ep = pool_ref[pl.ds(0, G), :]
        for j in range(1, NC * NS):
            ep = ep + pool_ref[pl.ds(j * G, G), :]
        glob_ref[...] = (_dot(ep, whe[...]) + _dot(npool_ref[...], whn[...])
                         + _dot(gf_ref[...], whu[...]) + bh[...])


_node_call = pl.pallas_call(
    _node_body,
    grid=(N // NBLK,),
    in_specs=[
        pl.BlockSpec((NBLK, D), lambda i: (i, 0)),
        pl.BlockSpec((NBLK, D), lambda i: (i, 0)),
        pl.BlockSpec((NBLK, D), lambda i: (i + N // NBLK, 0)),
        pl.BlockSpec((NBLK, D), lambda i: (i, 0)),
        pl.BlockSpec((NBLK, D), lambda i: (i + N // NBLK, 0)),
        pl.BlockSpec((1, 1, NBLK), lambda i: (i, 0, 0)),
        pl.BlockSpec((G, D), lambda i: (0, 0)),
        pl.BlockSpec((NC * NS * G, D), lambda i: (0, 0)),
        pl.BlockSpec((G, 16), lambda i: (0, 0)),
        pl.BlockSpec((D, D), lambda i: (0, 0)),
        pl.BlockSpec((D, D), lambda i: (0, 0)),
        pl.BlockSpec((D, D), lambda i: (0, 0)),
        pl.BlockSpec((1, D), lambda i: (0, 0)),
        pl.BlockSpec((D, D), lambda i: (0, 0)),
        pl.BlockSpec((D, D), lambda i: (0, 0)),
        pl.BlockSpec((16, D), lambda i: (0, 0)),
        pl.BlockSpec((1, D), lambda i: (0, 0)),
    ],
    out_specs=(
        pl.BlockSpec((NBLK, D), lambda i: (i, 0)),
        pl.BlockSpec((G, D), lambda i: (0, 0)),
    ),
    out_shape=(
        jax.ShapeDtypeStruct((N, D), jnp.float32),
        jax.ShapeDtypeStruct((G, D), jnp.float32),
    ),
    scratch_shapes=[pltpu.VMEM((G, D), jnp.float32)],
)


def kernel(node_features, edge_features, global_features, senders, receivers,
           edge_graph_ids, node_graph_ids,
           W_fe, b_fe, W_fs, b_fs, W_fr, b_fr, W_fu, b_fu,
           W_gn, b_gn, W_gin, b_gin, W_gout, b_gout, W_gu, b_gu,
           W_hn, b_hn, W_he, b_he, W_hu, b_hu):
    fe_all, fs_tab, fr_tab, gu_tab = _fe_call(
        edge_features, edge_graph_ids.reshape(E // EBLK, 1, EBLK),
        node_features, global_features,
        W_fe, b_fe.reshape(1, D), W_fs, b_fs.reshape(1, D),
        W_fr, b_fr.reshape(1, D), W_fu, b_fu.reshape(1, D),
        W_gu, b_gu.reshape(1, D))
    edges, aggin_part, aggout_part, pool_part = _sc_fused_pass(
        fe_all, fs_tab, fr_tab, senders, receivers, edge_graph_ids)
    bn_all = (b_gn + b_gin + b_gout).reshape(1, D)
    bh_all = (b_he + b_hn + b_hu).reshape(1, D)
    nodes, globals_out = _node_call(
        node_features, aggin_part, aggin_part, aggout_part, aggout_part,
        node_graph_ids.reshape(N // NBLK, 1, NBLK),
        gu_tab, pool_part, global_features,
        W_gn, W_gin, W_gout, bn_all, W_hn, W_he, W_hu, bh_all)
    return edges, nodes, globals_out
